# baseline (speedup 1.0000x reference)
import functools

import jax
import jax.numpy as jnp
from jax import lax
from jax.experimental import pallas as pl
from jax.experimental.pallas import tpu as pltpu
from jax.experimental.pallas import tpu_sc as plsc

N = 10000
E = 320000
FEATS = 128
HIDDEN = 64
C = 16
DEPTH = 10
HID2 = 147
HID2P = 256

NC = 2
NS = 16
NW = NC * NS
CHUNK = 128
K = 4
NSUP = 4
NCHT = 84
QCH = NCHT // NSUP
NCH = NCHT - K
EPAD = NW * NCH * CHUNK
NP = 10112
RPT = NP // NS

DIFF = [0.9 ** l for l in range(1, DEPTH + 1)]
DSUM = 1.0 + sum(DIFF)

_MESH = plsc.VectorSubcoreMesh(core_axis_name="c", subcore_axis_name="s")
_SC_PARAMS = pltpu.CompilerParams(use_tc_tiling_on_sc=False)


@functools.partial(
    pl.kernel,
    mesh=_MESH,
    out_type=jax.ShapeDtypeStruct((NC, NP, C), jnp.float32),
    scratch_types=[
        pltpu.VMEM((NSUP, QCH * CHUNK), jnp.int32),
        pltpu.VMEM((QCH * CHUNK, C), jnp.float32),
        pltpu.VMEM_SHARED((NP, C), jnp.float32),
    ],
    compiler_params=_SC_PARAMS,
)
def _sc_degree(dst_hbm, ones_hbm, zeros_hbm, part_hbm, dst_v, ones_v, acc):
    cid = lax.axis_index("c")
    sid = lax.axis_index("s")
    wid = sid * NC + cid
    pltpu.sync_copy(zeros_hbm, acc.at[pl.ds(sid * RPT, RPT)])
    pltpu.sync_copy(dst_hbm.at[wid], dst_v)
    pltpu.sync_copy(ones_hbm, ones_v)
    plsc.subcore_barrier()
    for q in range(NSUP):
        pltpu.sync_copy(ones_v, acc.at[dst_v.at[q]], add=True)
    plsc.subcore_barrier()
    pltpu.sync_copy(acc.at[pl.ds(sid * RPT, RPT)],
                    part_hbm.at[cid, pl.ds(sid * RPT, RPT)])


@functools.partial(
    pl.kernel,
    mesh=_MESH,
    out_type=jax.ShapeDtypeStruct((NC, NP, C), jnp.float32),
    scratch_types=[
        pltpu.VMEM((NSUP, QCH * CHUNK), jnp.int32),
        pltpu.VMEM((NSUP, QCH * CHUNK), jnp.int32),
        pltpu.VMEM((QCH * CHUNK, C), jnp.float32),
        pltpu.VMEM((QCH * CHUNK, C), jnp.float32),
        pltpu.VMEM_SHARED((NP, C), jnp.float32),
        pltpu.SemaphoreType.DMA,
        pltpu.SemaphoreType.DMA,
        pltpu.SemaphoreType.DMA,
        pltpu.SemaphoreType.DMA,
    ],
    compiler_params=_SC_PARAMS,
)
def _sc_conv(src_hbm, dst_hbm, y_hbm, zeros_hbm, part_hbm,
             src_v, dst_v, buf_a, buf_b, acc, sga, sgb, ssa, ssb):
    cid = lax.axis_index("c")
    sid = lax.axis_index("s")
    wid = sid * NC + cid
    pltpu.sync_copy(zeros_hbm, acc.at[pl.ds(sid * RPT, RPT)])
    pltpu.sync_copy(src_hbm.at[wid], src_v)
    pltpu.sync_copy(dst_hbm.at[wid], dst_v)
    plsc.subcore_barrier()

    bufs = (buf_a, buf_b)
    gsem = (sga, sgb)
    ssem = (ssa, ssb)

    def gather(q):
        b = q % 2
        return pltpu.async_copy(y_hbm.at[src_v.at[q]], bufs[b], gsem[b])

    def scatter(q):
        b = q % 2
        return pltpu.async_copy(bufs[b], acc.at[dst_v.at[q]], ssem[b],
                                add=True)

    for q in range(NSUP):
        gather(q).wait()
        scatter(q).wait()

    plsc.subcore_barrier()
    pltpu.sync_copy(acc.at[pl.ds(sid * RPT, RPT)],
                    part_hbm.at[cid, pl.ds(sid * RPT, RPT)])


GB = 8
BR = NP // GB

_row = pl.BlockSpec((BR, C), lambda i: (i, 0))
_rowx = pl.BlockSpec((BR, FEATS), lambda i: (i, 0))
_smem = pl.BlockSpec(memory_space=pltpu.SMEM)


def _full(shape):
    return pl.BlockSpec(shape, lambda i: tuple(0 for _ in shape))


def _dinv_body(part_ref, dinvb_ref, dinv2b_ref):
    i = pl.program_id(0)
    deg = part_ref[0] + part_ref[1] + 1.0
    dinv = lax.rsqrt(jnp.maximum(deg, 1.0))
    row = i * BR + lax.broadcasted_iota(jnp.int32, (BR, C), 0)
    dinv = dinv * (row < N).astype(jnp.float32)
    dinvb_ref[...] = dinv
    dinv2b_ref[...] = dinv * dinv


_tc_dinv = pl.pallas_call(
    _dinv_body,
    grid=(GB,),
    in_specs=[pl.BlockSpec((NC, BR, C), lambda i: (0, i, 0))],
    out_specs=(_row, _row),
    out_shape=(jax.ShapeDtypeStruct((NP, C), jnp.float32),
               jax.ShapeDtypeStruct((NP, C), jnp.float32)),
)


def _mlp_body(x_ref, w1_ref, b1_ref, w2_ref, b2_ref, dinvb_ref,
              cur_ref, y_ref):
    h1 = lax.dot_general(x_ref[...], w1_ref[...], (((1,), (1,)), ((), ())),
                         preferred_element_type=jnp.float32)
    h1 = jnp.maximum(h1 + b1_ref[...], 0.0)
    h = lax.dot_general(h1, w2_ref[...], (((1,), (1,)), ((), ())),
                        preferred_element_type=jnp.float32)
    h = h + b2_ref[...]
    cur_ref[...] = h
    y_ref[...] = h * dinvb_ref[...]


_tc_mlp = pl.pallas_call(
    _mlp_body,
    grid=(GB,),
    in_specs=[_rowx, _full((HIDDEN, FEATS)), _full((1, HIDDEN)),
              _full((C, HIDDEN)), _full((1, C)), _row],
    out_specs=(_row, _row),
    out_shape=(jax.ShapeDtypeStruct((NP, C), jnp.float32),
               jax.ShapeDtypeStruct((NP, C), jnp.float32)),
)


def _combine_body(part_ref, cur_ref, h0_ref, dinvb_ref, dinv2b_ref, d_ref,
                  ncur_ref, nh0_ref, ny_ref):
    s = part_ref[0] + part_ref[1]
    conv = dinvb_ref[...] * s + dinv2b_ref[...] * cur_ref[...]
    ncur_ref[...] = conv
    nh0_ref[...] = h0_ref[...] + d_ref[0, 0] * conv
    ny_ref[...] = dinvb_ref[...] * conv


_tc_combine = pl.pallas_call(
    _combine_body,
    grid=(GB,),
    in_specs=[pl.BlockSpec((NC, BR, C), lambda i: (0, i, 0)),
              _row, _row, _row, _row, _smem],
    out_specs=(_row, _row, _row),
    out_shape=(jax.ShapeDtypeStruct((NP, C), jnp.float32),
               jax.ShapeDtypeStruct((NP, C), jnp.float32),
               jax.ShapeDtypeStruct((NP, C), jnp.float32)),
)


def _attn_body(x_ref, h0_ref, a1x_ref, ba1_ref, u_ref, v_ref, a2_ref,
               ba2_ref, dinvb_ref, cur_ref, y_ref):
    xa = lax.dot_general(x_ref[...], a1x_ref[...], (((1,), (1,)), ((), ())),
                         preferred_element_type=jnp.float32)
    xa = xa + ba1_ref[...]
    z = h0_ref[...] * (1.0 / DSUM)
    ba2 = ba2_ref[0, 0]
    for c in range(C):
        t = jnp.maximum(xa + z[:, c:c + 1] * u_ref[...] + v_ref[c:c + 1, :],
                        0.0)
        sc = lax.dot_general(t, a2_ref[...], (((1,), (0,)), ((), ())),
                             preferred_element_type=jnp.float32)
        col = sc[:, 0:1] + ba2
        cur_ref[:, c:c + 1] = col
        y_ref[:, c:c + 1] = col * dinvb_ref[:, c:c + 1]


_tc_attn = pl.pallas_call(
    _attn_body,
    grid=(GB,),
    in_specs=[_rowx, _row, _full((HID2P, FEATS)), _full((1, HID2P)),
              _full((1, HID2P)), _full((C, HID2P)), _full((HID2P, 8)),
              _smem, _row],
    out_specs=(_row, _row),
    out_shape=(jax.ShapeDtypeStruct((NP, C), jnp.float32),
               jax.ShapeDtypeStruct((NP, C), jnp.float32)),
)


def _final_body(h0_ref, scl_ref, out_ref):
    out_ref[...] = h0_ref[...] * (scl_ref[0, 0] * (1.0 / DSUM))


_tc_final = pl.pallas_call(
    _final_body,
    grid=(GB,),
    in_specs=[_row, _smem],
    out_specs=_row,
    out_shape=jax.ShapeDtypeStruct((NP, C), jnp.float32),
)


def kernel(x, edges, classes, W1, b1, W2, b2, A1, ba1, A2, ba2):
    f32 = jnp.float32
    x = x.astype(f32)
    src = edges[0].astype(jnp.int32)
    dst = edges[1].astype(jnp.int32)

    pad = EPAD - E
    tail = jnp.full((NW, K, CHUNK), N, jnp.int32)
    src_t = jnp.concatenate([
        jnp.concatenate([src, jnp.full((pad,), N, jnp.int32)]).reshape(
            NW, NCH, CHUNK), tail], axis=1).reshape(NW, NSUP, QCH * CHUNK)
    dst_t = jnp.concatenate([
        jnp.concatenate([dst, jnp.full((pad,), N, jnp.int32)]).reshape(
            NW, NCH, CHUNK), tail], axis=1).reshape(NW, NSUP, QCH * CHUNK)

    xp = jnp.pad(x, ((0, NP - N), (0, 0)))
    zeros_rpt = jnp.zeros((RPT, C), f32)
    ones_chunk = jnp.ones((QCH * CHUNK, C), f32)

    part = _sc_degree(dst_t, ones_chunk, zeros_rpt)
    dinvb, dinv2b = _tc_dinv(part)

    b1r = b1.astype(f32).reshape(1, HIDDEN)
    b2r = b2.astype(f32).reshape(1, C)
    cur, y = _tc_mlp(xp, W1.astype(f32), b1r, W2.astype(f32), b2r, dinvb)
    d_arr = [jnp.full((1, 1), d, f32) for d in DIFF]
    h0 = cur
    for l in range(DEPTH):
        part = _sc_conv(src_t, dst_t, y, zeros_rpt)
        cur, h0, y = _tc_combine(part, cur, h0, dinvb, dinv2b, d_arr[l])

    A1f = A1.astype(f32)
    hp = HID2P - HID2
    a1x = jnp.pad(A1f[:, 1 + C:], ((0, hp), (0, 0)))
    ba1p = jnp.pad(ba1.astype(f32), (0, hp)).reshape(1, HID2P)
    up = jnp.pad(A1f[:, 0], (0, hp)).reshape(1, HID2P)
    vp = jnp.pad(A1f[:, 1:1 + C].T, ((0, 0), (0, hp)))
    a2p = jnp.pad(A2.astype(f32).T, ((0, hp), (0, 7)))
    ba2r = ba2.astype(f32).reshape(1, 1)

    cur, y = _tc_attn(xp, h0, a1x, ba1p, up, vp, a2p, ba2r, dinvb)
    h0 = cur
    for l in range(DEPTH):
        part = _sc_conv(src_t, dst_t, y, zeros_rpt)
        cur, h0, y = _tc_combine(part, cur, h0, dinvb, dinv2b, d_arr[l])

    scl = (jnp.asarray(classes, f32) / C).reshape(1, 1)
    out = _tc_final(h0, scl)
    return out[:N]


# speedup vs baseline: 1.2013x; 1.2013x over previous
import functools

import jax
import jax.numpy as jnp
from jax import lax
from jax.experimental import pallas as pl
from jax.experimental.pallas import tpu as pltpu
from jax.experimental.pallas import tpu_sc as plsc

N = 10000
E = 320000
FEATS = 128
HIDDEN = 64
C = 16
DEPTH = 10
HID2 = 147
HID2P = 256

NC = 2
NS = 16
NW = NC * NS
CHUNK = 128
K = 4
NCHT = 84
NCH = NCHT - K
EPAD = NW * NCH * CHUNK
NP = 10112
RPT = NP // NS

DIFF = [0.9 ** l for l in range(1, DEPTH + 1)]
DSUM = 1.0 + sum(DIFF)

_MESH = plsc.VectorSubcoreMesh(core_axis_name="c", subcore_axis_name="s")
_SC_PARAMS = pltpu.CompilerParams(use_tc_tiling_on_sc=False)


@functools.partial(
    pl.kernel,
    mesh=_MESH,
    out_type=jax.ShapeDtypeStruct((NC, NP, C), jnp.float32),
    scratch_types=[
        pltpu.VMEM((NCHT, CHUNK), jnp.int32),
        pltpu.VMEM((CHUNK, C), jnp.float32),
        pltpu.VMEM_SHARED((NP, C), jnp.float32),
    ],
    compiler_params=_SC_PARAMS,
)
def _sc_degree(dst_hbm, ones_hbm, zeros_hbm, part_hbm, dst_v, ones_v, acc):
    cid = lax.axis_index("c")
    sid = lax.axis_index("s")
    wid = sid * NC + cid
    pltpu.sync_copy(zeros_hbm, acc.at[pl.ds(sid * RPT, RPT)])
    pltpu.sync_copy(dst_hbm.at[wid], dst_v)
    pltpu.sync_copy(ones_hbm, ones_v)
    plsc.subcore_barrier()

    def body(j, carry):
        pltpu.sync_copy(ones_v, acc.at[dst_v.at[j]], add=True)
        return carry

    lax.fori_loop(0, NCHT, body, 0)
    plsc.subcore_barrier()
    pltpu.sync_copy(acc.at[pl.ds(sid * RPT, RPT)],
                    part_hbm.at[cid, pl.ds(sid * RPT, RPT)])


@functools.partial(
    pl.kernel,
    mesh=_MESH,
    out_type=jax.ShapeDtypeStruct((NC, NP, C), jnp.float32),
    scratch_types=[
        pltpu.VMEM((NCHT, CHUNK), jnp.int32),
        pltpu.VMEM((NCHT, CHUNK), jnp.int32),
        pltpu.VMEM((CHUNK, C), jnp.float32),
        pltpu.VMEM((CHUNK, C), jnp.float32),
        pltpu.VMEM_SHARED((NP, C), jnp.float32),
        pltpu.SemaphoreType.DMA,
        pltpu.SemaphoreType.DMA,
    ],
    compiler_params=_SC_PARAMS,
)
def _sc_conv(src_hbm, dst_hbm, y_hbm, zeros_hbm, part_hbm,
             src_v, dst_v, buf0, buf1, acc, sg0, sg1):
    cid = lax.axis_index("c")
    sid = lax.axis_index("s")
    wid = sid * NC + cid
    pltpu.sync_copy(zeros_hbm, acc.at[pl.ds(sid * RPT, RPT)])
    pltpu.sync_copy(src_hbm.at[wid], src_v)
    pltpu.sync_copy(dst_hbm.at[wid], dst_v)
    plsc.subcore_barrier()

    def fire_g(j, buf, sem):
        pltpu.async_copy(y_hbm.at[src_v.at[j]], buf, sem)

    def wait_g(j, buf, sem):
        pltpu.make_async_copy(y_hbm.at[src_v.at[j]], buf, sem).wait()

    fire_g(0, buf0, sg0)

    def body(t, carry):
        j = 2 * t
        wait_g(j, buf0, sg0)
        fire_g(j + 1, buf1, sg1)
        pltpu.sync_copy(buf0, acc.at[dst_v.at[j]], add=True)
        wait_g(j + 1, buf1, sg1)
        fire_g(j + 2, buf0, sg0)
        pltpu.sync_copy(buf1, acc.at[dst_v.at[j + 1]], add=True)
        return carry

    lax.fori_loop(0, NCH // 2, body, 0)
    wait_g(NCH, buf0, sg0)
    plsc.subcore_barrier()
    pltpu.sync_copy(acc.at[pl.ds(sid * RPT, RPT)],
                    part_hbm.at[cid, pl.ds(sid * RPT, RPT)])


GB = 8
BR = NP // GB

_row = pl.BlockSpec((BR, C), lambda i: (i, 0))
_rowx = pl.BlockSpec((BR, FEATS), lambda i: (i, 0))
_smem = pl.BlockSpec(memory_space=pltpu.SMEM)


def _full(shape):
    return pl.BlockSpec(shape, lambda i: tuple(0 for _ in shape))


def _dinv_body(part_ref, dinvb_ref, dinv2b_ref):
    i = pl.program_id(0)
    deg = part_ref[0] + part_ref[1] + 1.0
    dinv = lax.rsqrt(jnp.maximum(deg, 1.0))
    row = i * BR + lax.broadcasted_iota(jnp.int32, (BR, C), 0)
    dinv = dinv * (row < N).astype(jnp.float32)
    dinvb_ref[...] = dinv
    dinv2b_ref[...] = dinv * dinv


_tc_dinv = pl.pallas_call(
    _dinv_body,
    grid=(GB,),
    in_specs=[pl.BlockSpec((NC, BR, C), lambda i: (0, i, 0))],
    out_specs=(_row, _row),
    out_shape=(jax.ShapeDtypeStruct((NP, C), jnp.float32),
               jax.ShapeDtypeStruct((NP, C), jnp.float32)),
)


def _mlp_body(x_ref, w1_ref, b1_ref, w2_ref, b2_ref, dinvb_ref,
              cur_ref, y_ref):
    h1 = lax.dot_general(x_ref[...], w1_ref[...], (((1,), (1,)), ((), ())),
                         preferred_element_type=jnp.float32)
    h1 = jnp.maximum(h1 + b1_ref[...], 0.0)
    h = lax.dot_general(h1, w2_ref[...], (((1,), (1,)), ((), ())),
                        preferred_element_type=jnp.float32)
    h = h + b2_ref[...]
    cur_ref[...] = h
    y_ref[...] = h * dinvb_ref[...]


_tc_mlp = pl.pallas_call(
    _mlp_body,
    grid=(GB,),
    in_specs=[_rowx, _full((HIDDEN, FEATS)), _full((1, HIDDEN)),
              _full((C, HIDDEN)), _full((1, C)), _row],
    out_specs=(_row, _row),
    out_shape=(jax.ShapeDtypeStruct((NP, C), jnp.float32),
               jax.ShapeDtypeStruct((NP, C), jnp.float32)),
)


def _combine_body(part_ref, cur_ref, h0_ref, dinvb_ref, dinv2b_ref, d_ref,
                  ncur_ref, nh0_ref, ny_ref):
    s = part_ref[0] + part_ref[1]
    conv = dinvb_ref[...] * s + dinv2b_ref[...] * cur_ref[...]
    ncur_ref[...] = conv
    nh0_ref[...] = h0_ref[...] + d_ref[0, 0] * conv
    ny_ref[...] = dinvb_ref[...] * conv


_tc_combine = pl.pallas_call(
    _combine_body,
    grid=(GB,),
    in_specs=[pl.BlockSpec((NC, BR, C), lambda i: (0, i, 0)),
              _row, _row, _row, _row, _smem],
    out_specs=(_row, _row, _row),
    out_shape=(jax.ShapeDtypeStruct((NP, C), jnp.float32),
               jax.ShapeDtypeStruct((NP, C), jnp.float32),
               jax.ShapeDtypeStruct((NP, C), jnp.float32)),
)


def _attn_body(x_ref, h0_ref, a1x_ref, ba1_ref, u_ref, v_ref, a2_ref,
               ba2_ref, dinvb_ref, cur_ref, y_ref):
    xa = lax.dot_general(x_ref[...], a1x_ref[...], (((1,), (1,)), ((), ())),
                         preferred_element_type=jnp.float32)
    xa = xa + ba1_ref[...]
    z = h0_ref[...] * (1.0 / DSUM)
    ba2 = ba2_ref[0, 0]
    for c in range(C):
        t = jnp.maximum(xa + z[:, c:c + 1] * u_ref[...] + v_ref[c:c + 1, :],
                        0.0)
        sc = lax.dot_general(t, a2_ref[...], (((1,), (0,)), ((), ())),
                             preferred_element_type=jnp.float32)
        col = sc[:, 0:1] + ba2
        cur_ref[:, c:c + 1] = col
        y_ref[:, c:c + 1] = col * dinvb_ref[:, c:c + 1]


_tc_attn = pl.pallas_call(
    _attn_body,
    grid=(GB,),
    in_specs=[_rowx, _row, _full((HID2P, FEATS)), _full((1, HID2P)),
              _full((1, HID2P)), _full((C, HID2P)), _full((HID2P, 8)),
              _smem, _row],
    out_specs=(_row, _row),
    out_shape=(jax.ShapeDtypeStruct((NP, C), jnp.float32),
               jax.ShapeDtypeStruct((NP, C), jnp.float32)),
)


def _final_body(h0_ref, scl_ref, out_ref):
    out_ref[...] = h0_ref[...] * (scl_ref[0, 0] * (1.0 / DSUM))


_tc_final = pl.pallas_call(
    _final_body,
    grid=(GB,),
    in_specs=[_row, _smem],
    out_specs=_row,
    out_shape=jax.ShapeDtypeStruct((NP, C), jnp.float32),
)


def kernel(x, edges, classes, W1, b1, W2, b2, A1, ba1, A2, ba2):
    f32 = jnp.float32
    x = x.astype(f32)
    src = edges[0].astype(jnp.int32)
    dst = edges[1].astype(jnp.int32)

    pad = EPAD - E
    tail = jnp.full((NW, K, CHUNK), N, jnp.int32)
    src_t = jnp.concatenate([
        jnp.concatenate([src, jnp.full((pad,), N, jnp.int32)]).reshape(
            NW, NCH, CHUNK), tail], axis=1)
    dst_t = jnp.concatenate([
        jnp.concatenate([dst, jnp.full((pad,), N, jnp.int32)]).reshape(
            NW, NCH, CHUNK), tail], axis=1)

    xp = jnp.pad(x, ((0, NP - N), (0, 0)))
    zeros_rpt = jnp.zeros((RPT, C), f32)
    ones_chunk = jnp.ones((CHUNK, C), f32)

    part = _sc_degree(dst_t, ones_chunk, zeros_rpt)
    dinvb, dinv2b = _tc_dinv(part)

    b1r = b1.astype(f32).reshape(1, HIDDEN)
    b2r = b2.astype(f32).reshape(1, C)
    cur, y = _tc_mlp(xp, W1.astype(f32), b1r, W2.astype(f32), b2r, dinvb)
    d_arr = [jnp.full((1, 1), d, f32) for d in DIFF]
    h0 = cur
    for l in range(DEPTH):
        part = _sc_conv(src_t, dst_t, y, zeros_rpt)
        cur, h0, y = _tc_combine(part, cur, h0, dinvb, dinv2b, d_arr[l])

    A1f = A1.astype(f32)
    hp = HID2P - HID2
    a1x = jnp.pad(A1f[:, 1 + C:], ((0, hp), (0, 0)))
    ba1p = jnp.pad(ba1.astype(f32), (0, hp)).reshape(1, HID2P)
    up = jnp.pad(A1f[:, 0], (0, hp)).reshape(1, HID2P)
    vp = jnp.pad(A1f[:, 1:1 + C].T, ((0, 0), (0, hp)))
    a2p = jnp.pad(A2.astype(f32).T, ((0, hp), (0, 7)))
    ba2r = ba2.astype(f32).reshape(1, 1)

    cur, y = _tc_attn(xp, h0, a1x, ba1p, up, vp, a2p, ba2r, dinvb)
    h0 = cur
    for l in range(DEPTH):
        part = _sc_conv(src_t, dst_t, y, zeros_rpt)
        cur, h0, y = _tc_combine(part, cur, h0, dinvb, dinv2b, d_arr[l])

    scl = (jnp.asarray(classes, f32) / C).reshape(1, 1)
    out = _tc_final(h0, scl)
    return out[:N]


# speedup vs baseline: 2.7508x; 2.2899x over previous
import functools

import jax
import jax.numpy as jnp
from jax import lax
from jax.experimental import pallas as pl
from jax.experimental.pallas import tpu as pltpu
from jax.experimental.pallas import tpu_sc as plsc

N = 10000
E = 320000
FEATS = 128
HIDDEN = 64
C = 16
DEPTH = 10
HID2 = 147
HID2P = 256

NC = 2
NS = 16
NW = NC * NS
CHUNK = 128
K = 4
NCHT = 84
NCH = NCHT - K
EPAD = NW * NCH * CHUNK
NP = 10112
RPT = NP // NS

DIFF = [0.9 ** l for l in range(1, DEPTH + 1)]
DSUM = 1.0 + sum(DIFF)

_MESH = plsc.VectorSubcoreMesh(core_axis_name="c", subcore_axis_name="s")
_SC_PARAMS = pltpu.CompilerParams(use_tc_tiling_on_sc=False)


@functools.partial(
    pl.kernel,
    mesh=_MESH,
    out_type=jax.ShapeDtypeStruct((NC, NP, C), jnp.float32),
    scratch_types=[
        pltpu.VMEM((NCHT, CHUNK), jnp.int32),
        pltpu.VMEM((CHUNK, C), jnp.float32),
        pltpu.VMEM_SHARED((NP, C), jnp.float32),
    ],
    compiler_params=_SC_PARAMS,
)
def _sc_degree(dst_hbm, ones_hbm, zeros_hbm, part_hbm, dst_v, ones_v, acc):
    cid = lax.axis_index("c")
    sid = lax.axis_index("s")
    wid = sid * NC + cid
    pltpu.sync_copy(zeros_hbm, acc.at[pl.ds(sid * RPT, RPT)])
    pltpu.sync_copy(dst_hbm.at[wid], dst_v)
    pltpu.sync_copy(ones_hbm, ones_v)
    plsc.subcore_barrier()

    def body(j, carry):
        pltpu.sync_copy(ones_v, acc.at[dst_v.at[j]], add=True)
        return carry

    lax.fori_loop(0, NCHT, body, 0)
    plsc.subcore_barrier()
    pltpu.sync_copy(acc.at[pl.ds(sid * RPT, RPT)],
                    part_hbm.at[cid, pl.ds(sid * RPT, RPT)])


@functools.partial(
    pl.kernel,
    mesh=_MESH,
    out_type=jax.ShapeDtypeStruct((NC, NP, C), jnp.float32),
    scratch_types=[
        pltpu.VMEM((NCHT, CHUNK), jnp.int32),
        pltpu.VMEM((NCHT, CHUNK), jnp.int32),
        pltpu.VMEM((CHUNK, C), jnp.float32),
        pltpu.VMEM_SHARED((NP, C), jnp.float32),
        pltpu.SemaphoreType.DMA,
    ],
    compiler_params=_SC_PARAMS,
)
def _sc_conv(src_hbm, dst_hbm, y_hbm, zeros_hbm, part_hbm,
             src_v, dst_v, rows_v, acc, sem):
    cid = lax.axis_index("c")
    sid = lax.axis_index("s")
    wid = sid * NC + cid
    pltpu.sync_copy(zeros_hbm, acc.at[pl.ds(sid * RPT, RPT)])
    pltpu.sync_copy(src_hbm.at[wid], src_v)
    pltpu.sync_copy(dst_hbm.at[wid], dst_v)
    plsc.subcore_barrier()

    def body(j, carry):
        pltpu.async_copy(y_hbm.at[src_v.at[j]], rows_v, sem).wait()
        pltpu.sync_copy(rows_v, acc.at[dst_v.at[j]], add=True)
        return carry

    lax.fori_loop(0, NCH, body, 0)
    plsc.subcore_barrier()
    pltpu.sync_copy(acc.at[pl.ds(sid * RPT, RPT)],
                    part_hbm.at[cid, pl.ds(sid * RPT, RPT)])


HRPT = RPT // 2


@functools.partial(
    pl.kernel,
    mesh=_MESH,
    out_type=(jax.ShapeDtypeStruct((NC, NP, C), jnp.float32),
              jax.ShapeDtypeStruct((NP, C), jnp.float32),
              jax.ShapeDtypeStruct((NP, C), jnp.float32)),
    scratch_types=[
        pltpu.VMEM((NCHT, CHUNK), jnp.int32),
        pltpu.VMEM((NCHT, CHUNK), jnp.int32),
        pltpu.VMEM((CHUNK, C), jnp.float32),
        pltpu.VMEM((RPT, C), jnp.float32),
        pltpu.VMEM((RPT, C), jnp.float32),
        pltpu.VMEM((RPT, C), jnp.float32),
        pltpu.VMEM((RPT, C), jnp.float32),
        pltpu.VMEM((RPT, C), jnp.float32),
        pltpu.VMEM((RPT, C), jnp.float32),
        pltpu.VMEM((RPT, C), jnp.float32),
        pltpu.VMEM((C,), jnp.float32),
        pltpu.VMEM_SHARED((NP, C), jnp.float32),
        pltpu.VMEM_SHARED((NP, C), jnp.float32),
        pltpu.SemaphoreType.DMA,
    ],
    compiler_params=_SC_PARAMS,
)
def _sc_round(src_hbm, dst_hbm, part_in, cur_in, h0_in, dinvb_hbm,
              dinv2b_hbm, zeros_hbm, dvec_hbm,
              part_out, cur_out, h0_out,
              src_v, dst_v, rows_v, p0_v, p1_v, cur_v, h0_v, db_v, d2_v,
              yb_v, dd_v, y_sh, acc, sem):
    cid = lax.axis_index("c")
    sid = lax.axis_index("s")
    wid = sid * NC + cid
    base = sid * RPT
    pltpu.sync_copy(zeros_hbm, acc.at[pl.ds(base, RPT)])
    pltpu.sync_copy(src_hbm.at[wid], src_v)
    pltpu.sync_copy(dst_hbm.at[wid], dst_v)
    pltpu.sync_copy(part_in.at[0, pl.ds(base, RPT)], p0_v)
    pltpu.sync_copy(part_in.at[1, pl.ds(base, RPT)], p1_v)
    pltpu.sync_copy(cur_in.at[pl.ds(base, RPT)], cur_v)
    pltpu.sync_copy(h0_in.at[pl.ds(base, RPT)], h0_v)
    pltpu.sync_copy(dinvb_hbm.at[pl.ds(base, RPT)], db_v)
    pltpu.sync_copy(dinv2b_hbm.at[pl.ds(base, RPT)], d2_v)
    pltpu.sync_copy(dvec_hbm, dd_v)
    dd = dd_v[...]

    def rowbody(i, carry):
        conv = db_v[i] * (p0_v[i] + p1_v[i]) + d2_v[i] * cur_v[i]
        cur_v[i] = conv
        h0_v[i] = h0_v[i] + dd * conv
        yb_v[i] = db_v[i] * conv
        return carry

    lax.fori_loop(0, RPT, rowbody, 0)
    pltpu.sync_copy(yb_v, y_sh.at[pl.ds(base, RPT)])
    half = cid * HRPT
    pltpu.sync_copy(cur_v.at[pl.ds(half, HRPT)],
                    cur_out.at[pl.ds(base + half, HRPT)])
    pltpu.sync_copy(h0_v.at[pl.ds(half, HRPT)],
                    h0_out.at[pl.ds(base + half, HRPT)])
    plsc.subcore_barrier()

    def body(j, carry):
        pltpu.async_copy(y_sh.at[src_v.at[j]], rows_v, sem).wait()
        pltpu.sync_copy(rows_v, acc.at[dst_v.at[j]], add=True)
        return carry

    lax.fori_loop(0, NCH, body, 0)
    plsc.subcore_barrier()
    pltpu.sync_copy(acc.at[pl.ds(base, RPT)],
                    part_out.at[cid, pl.ds(base, RPT)])


GB = 8
BR = NP // GB

_row = pl.BlockSpec((BR, C), lambda i: (i, 0))
_rowx = pl.BlockSpec((BR, FEATS), lambda i: (i, 0))
_smem = pl.BlockSpec(memory_space=pltpu.SMEM)


def _full(shape):
    return pl.BlockSpec(shape, lambda i: tuple(0 for _ in shape))


def _dinv_body(part_ref, dinvb_ref, dinv2b_ref):
    i = pl.program_id(0)
    deg = part_ref[0] + part_ref[1] + 1.0
    dinv = lax.rsqrt(jnp.maximum(deg, 1.0))
    row = i * BR + lax.broadcasted_iota(jnp.int32, (BR, C), 0)
    dinv = dinv * (row < N).astype(jnp.float32)
    dinvb_ref[...] = dinv
    dinv2b_ref[...] = dinv * dinv


_tc_dinv = pl.pallas_call(
    _dinv_body,
    grid=(GB,),
    in_specs=[pl.BlockSpec((NC, BR, C), lambda i: (0, i, 0))],
    out_specs=(_row, _row),
    out_shape=(jax.ShapeDtypeStruct((NP, C), jnp.float32),
               jax.ShapeDtypeStruct((NP, C), jnp.float32)),
)


def _mlp_body(x_ref, w1_ref, b1_ref, w2_ref, b2_ref, dinvb_ref,
              cur_ref, y_ref):
    h1 = lax.dot_general(x_ref[...], w1_ref[...], (((1,), (1,)), ((), ())),
                         preferred_element_type=jnp.float32)
    h1 = jnp.maximum(h1 + b1_ref[...], 0.0)
    h = lax.dot_general(h1, w2_ref[...], (((1,), (1,)), ((), ())),
                        preferred_element_type=jnp.float32)
    h = h + b2_ref[...]
    cur_ref[...] = h
    y_ref[...] = h * dinvb_ref[...]


_tc_mlp = pl.pallas_call(
    _mlp_body,
    grid=(GB,),
    in_specs=[_rowx, _full((HIDDEN, FEATS)), _full((1, HIDDEN)),
              _full((C, HIDDEN)), _full((1, C)), _row],
    out_specs=(_row, _row),
    out_shape=(jax.ShapeDtypeStruct((NP, C), jnp.float32),
               jax.ShapeDtypeStruct((NP, C), jnp.float32)),
)


def _combine_body(part_ref, cur_ref, h0_ref, dinvb_ref, dinv2b_ref, d_ref,
                  ncur_ref, nh0_ref, ny_ref):
    s = part_ref[0] + part_ref[1]
    conv = dinvb_ref[...] * s + dinv2b_ref[...] * cur_ref[...]
    ncur_ref[...] = conv
    nh0_ref[...] = h0_ref[...] + d_ref[0, 0] * conv
    ny_ref[...] = dinvb_ref[...] * conv


_tc_combine = pl.pallas_call(
    _combine_body,
    grid=(GB,),
    in_specs=[pl.BlockSpec((NC, BR, C), lambda i: (0, i, 0)),
              _row, _row, _row, _row, _smem],
    out_specs=(_row, _row, _row),
    out_shape=(jax.ShapeDtypeStruct((NP, C), jnp.float32),
               jax.ShapeDtypeStruct((NP, C), jnp.float32),
               jax.ShapeDtypeStruct((NP, C), jnp.float32)),
)


def _attn_body(x_ref, h0_ref, a1x_ref, ba1_ref, u_ref, v_ref, a2_ref,
               ba2_ref, dinvb_ref, cur_ref, y_ref):
    xa = lax.dot_general(x_ref[...], a1x_ref[...], (((1,), (1,)), ((), ())),
                         preferred_element_type=jnp.float32)
    xa = xa + ba1_ref[...]
    z = h0_ref[...] * (1.0 / DSUM)
    ba2 = ba2_ref[0, 0]
    for c in range(C):
        t = jnp.maximum(xa + z[:, c:c + 1] * u_ref[...] + v_ref[c:c + 1, :],
                        0.0)
        sc = lax.dot_general(t, a2_ref[...], (((1,), (0,)), ((), ())),
                             preferred_element_type=jnp.float32)
        col = sc[:, 0:1] + ba2
        cur_ref[:, c:c + 1] = col
        y_ref[:, c:c + 1] = col * dinvb_ref[:, c:c + 1]


_tc_attn = pl.pallas_call(
    _attn_body,
    grid=(GB,),
    in_specs=[_rowx, _row, _full((HID2P, FEATS)), _full((1, HID2P)),
              _full((1, HID2P)), _full((C, HID2P)), _full((HID2P, 8)),
              _smem, _row],
    out_specs=(_row, _row),
    out_shape=(jax.ShapeDtypeStruct((NP, C), jnp.float32),
               jax.ShapeDtypeStruct((NP, C), jnp.float32)),
)


def _final_body(h0_ref, scl_ref, out_ref):
    out_ref[...] = h0_ref[...] * (scl_ref[0, 0] * (1.0 / DSUM))


_tc_final = pl.pallas_call(
    _final_body,
    grid=(GB,),
    in_specs=[_row, _smem],
    out_specs=_row,
    out_shape=jax.ShapeDtypeStruct((NP, C), jnp.float32),
)


def kernel(x, edges, classes, W1, b1, W2, b2, A1, ba1, A2, ba2):
    f32 = jnp.float32
    x = x.astype(f32)
    src = edges[0].astype(jnp.int32)
    dst = edges[1].astype(jnp.int32)

    pad = EPAD - E
    tail = jnp.full((NW, K, CHUNK), N, jnp.int32)
    src_t = jnp.concatenate([
        jnp.concatenate([src, jnp.full((pad,), N, jnp.int32)]).reshape(
            NW, NCH, CHUNK), tail], axis=1)
    dst_t = jnp.concatenate([
        jnp.concatenate([dst, jnp.full((pad,), N, jnp.int32)]).reshape(
            NW, NCH, CHUNK), tail], axis=1)

    xp = jnp.pad(x, ((0, NP - N), (0, 0)))
    zeros_rpt = jnp.zeros((RPT, C), f32)
    ones_chunk = jnp.ones((CHUNK, C), f32)

    part = _sc_degree(dst_t, ones_chunk, zeros_rpt)
    dinvb, dinv2b = _tc_dinv(part)

    b1r = b1.astype(f32).reshape(1, HIDDEN)
    b2r = b2.astype(f32).reshape(1, C)
    cur, y = _tc_mlp(xp, W1.astype(f32), b1r, W2.astype(f32), b2r, dinvb)
    d_arr = [jnp.full((1, 1), d, f32) for d in DIFF]
    d_vec = [jnp.full((C,), d, f32) for d in DIFF]

    def diffuse(cur, y):
        h0 = cur
        part = _sc_conv(src_t, dst_t, y, zeros_rpt)
        for l in range(DEPTH - 1):
            part, cur, h0 = _sc_round(src_t, dst_t, part, cur, h0, dinvb,
                                      dinv2b, zeros_rpt, d_vec[l])
        cur, h0, y = _tc_combine(part, cur, h0, dinvb, dinv2b,
                                 d_arr[DEPTH - 1])
        return h0

    h0 = diffuse(cur, y)

    A1f = A1.astype(f32)
    hp = HID2P - HID2
    a1x = jnp.pad(A1f[:, 1 + C:], ((0, hp), (0, 0)))
    ba1p = jnp.pad(ba1.astype(f32), (0, hp)).reshape(1, HID2P)
    up = jnp.pad(A1f[:, 0], (0, hp)).reshape(1, HID2P)
    vp = jnp.pad(A1f[:, 1:1 + C].T, ((0, 0), (0, hp)))
    a2p = jnp.pad(A2.astype(f32).T, ((0, hp), (0, 7)))
    ba2r = ba2.astype(f32).reshape(1, 1)

    cur, y = _tc_attn(xp, h0, a1x, ba1p, up, vp, a2p, ba2r, dinvb)
    h0 = diffuse(cur, y)

    scl = (jnp.asarray(classes, f32) / C).reshape(1, 1)
    out = _tc_final(h0, scl)
    return out[:N]


# speedup vs baseline: 2.7664x; 1.0057x over previous
import functools

import jax
import jax.numpy as jnp
from jax import lax
from jax.experimental import pallas as pl
from jax.experimental.pallas import tpu as pltpu
from jax.experimental.pallas import tpu_sc as plsc

N = 10000
E = 320000
FEATS = 128
HIDDEN = 64
C = 16
DEPTH = 10
HID2 = 147
HID2P = 256

NC = 2
NS = 16
NW = NC * NS
CHUNK = 128
K = 4
NCHT = 84
NCH = NCHT - K
EPAD = NW * NCH * CHUNK
NP = 10112
RPT = NP // NS

DIFF = [0.9 ** l for l in range(1, DEPTH + 1)]
DSUM = 1.0 + sum(DIFF)

_MESH = plsc.VectorSubcoreMesh(core_axis_name="c", subcore_axis_name="s")
_SC_PARAMS = pltpu.CompilerParams(use_tc_tiling_on_sc=False)


@functools.partial(
    pl.kernel,
    mesh=_MESH,
    out_type=jax.ShapeDtypeStruct((NC, NP, C), jnp.float32),
    scratch_types=[
        pltpu.VMEM((NCHT, CHUNK), jnp.int32),
        pltpu.VMEM((CHUNK, C), jnp.float32),
        pltpu.VMEM_SHARED((NP, C), jnp.float32),
    ],
    compiler_params=_SC_PARAMS,
)
def _sc_degree(dst_hbm, ones_hbm, zeros_hbm, part_hbm, dst_v, ones_v, acc):
    cid = lax.axis_index("c")
    sid = lax.axis_index("s")
    wid = sid * NC + cid
    pltpu.sync_copy(zeros_hbm, acc.at[pl.ds(sid * RPT, RPT)])
    pltpu.sync_copy(dst_hbm.at[wid], dst_v)
    pltpu.sync_copy(ones_hbm, ones_v)
    plsc.subcore_barrier()

    def body(j, carry):
        pltpu.sync_copy(ones_v, acc.at[dst_v.at[j]], add=True)
        return carry

    lax.fori_loop(0, NCHT, body, 0)
    plsc.subcore_barrier()
    pltpu.sync_copy(acc.at[pl.ds(sid * RPT, RPT)],
                    part_hbm.at[cid, pl.ds(sid * RPT, RPT)])


@functools.partial(
    pl.kernel,
    mesh=_MESH,
    out_type=jax.ShapeDtypeStruct((NC, NP, C), jnp.float32),
    scratch_types=[
        pltpu.VMEM((NCHT, CHUNK), jnp.int32),
        pltpu.VMEM((NCHT, CHUNK), jnp.int32),
        pltpu.VMEM((CHUNK, C), jnp.float32),
        pltpu.VMEM_SHARED((NP, C), jnp.float32),
        pltpu.VMEM_SHARED((NP, C), jnp.float32),
        pltpu.SemaphoreType.DMA,
    ],
    compiler_params=_SC_PARAMS,
)
def _sc_conv(src_hbm, dst_hbm, y_hbm, zeros_hbm, part_hbm,
             src_v, dst_v, rows_v, y_sh, acc, sem):
    cid = lax.axis_index("c")
    sid = lax.axis_index("s")
    wid = sid * NC + cid
    base = sid * RPT
    pltpu.sync_copy(zeros_hbm, acc.at[pl.ds(base, RPT)])
    pltpu.sync_copy(y_hbm.at[pl.ds(base, RPT)], y_sh.at[pl.ds(base, RPT)])
    pltpu.sync_copy(src_hbm.at[wid], src_v)
    pltpu.sync_copy(dst_hbm.at[wid], dst_v)
    plsc.subcore_barrier()

    def body(j, carry):
        pltpu.async_copy(y_sh.at[src_v.at[j]], rows_v, sem).wait()
        pltpu.sync_copy(rows_v, acc.at[dst_v.at[j]], add=True)
        return carry

    lax.fori_loop(0, NCH, body, 0)
    plsc.subcore_barrier()
    pltpu.sync_copy(acc.at[pl.ds(base, RPT)],
                    part_hbm.at[cid, pl.ds(base, RPT)])


HRPT = RPT // 2


@functools.partial(
    pl.kernel,
    mesh=_MESH,
    out_type=(jax.ShapeDtypeStruct((NC, NP, C), jnp.float32),
              jax.ShapeDtypeStruct((NP, C), jnp.float32),
              jax.ShapeDtypeStruct((NP, C), jnp.float32)),
    scratch_types=[
        pltpu.VMEM((NCHT, CHUNK), jnp.int32),
        pltpu.VMEM((NCHT, CHUNK), jnp.int32),
        pltpu.VMEM((CHUNK, C), jnp.float32),
        pltpu.VMEM((RPT, C), jnp.float32),
        pltpu.VMEM((RPT, C), jnp.float32),
        pltpu.VMEM((RPT, C), jnp.float32),
        pltpu.VMEM((RPT, C), jnp.float32),
        pltpu.VMEM((RPT, C), jnp.float32),
        pltpu.VMEM((RPT, C), jnp.float32),
        pltpu.VMEM((RPT, C), jnp.float32),
        pltpu.VMEM((C,), jnp.float32),
        pltpu.VMEM_SHARED((NP, C), jnp.float32),
        pltpu.VMEM_SHARED((NP, C), jnp.float32),
        pltpu.SemaphoreType.DMA,
    ],
    compiler_params=_SC_PARAMS,
)
def _sc_round(src_hbm, dst_hbm, part_in, cur_in, h0_in, dinvb_hbm,
              dinv2b_hbm, zeros_hbm, dvec_hbm,
              part_out, cur_out, h0_out,
              src_v, dst_v, rows_v, p0_v, p1_v, cur_v, h0_v, db_v, d2_v,
              yb_v, dd_v, y_sh, acc, sem):
    cid = lax.axis_index("c")
    sid = lax.axis_index("s")
    wid = sid * NC + cid
    base = sid * RPT
    pltpu.sync_copy(zeros_hbm, acc.at[pl.ds(base, RPT)])
    pltpu.sync_copy(src_hbm.at[wid], src_v)
    pltpu.sync_copy(dst_hbm.at[wid], dst_v)
    pltpu.sync_copy(part_in.at[0, pl.ds(base, RPT)], p0_v)
    pltpu.sync_copy(part_in.at[1, pl.ds(base, RPT)], p1_v)
    pltpu.sync_copy(cur_in.at[pl.ds(base, RPT)], cur_v)
    pltpu.sync_copy(h0_in.at[pl.ds(base, RPT)], h0_v)
    pltpu.sync_copy(dinvb_hbm.at[pl.ds(base, RPT)], db_v)
    pltpu.sync_copy(dinv2b_hbm.at[pl.ds(base, RPT)], d2_v)
    pltpu.sync_copy(dvec_hbm, dd_v)
    dd = dd_v[...]

    def rowbody(i, carry):
        conv = db_v[i] * (p0_v[i] + p1_v[i]) + d2_v[i] * cur_v[i]
        cur_v[i] = conv
        h0_v[i] = h0_v[i] + dd * conv
        yb_v[i] = db_v[i] * conv
        return carry

    lax.fori_loop(0, RPT, rowbody, 0, unroll=4)
    pltpu.sync_copy(yb_v, y_sh.at[pl.ds(base, RPT)])
    half = cid * HRPT
    pltpu.sync_copy(cur_v.at[pl.ds(half, HRPT)],
                    cur_out.at[pl.ds(base + half, HRPT)])
    pltpu.sync_copy(h0_v.at[pl.ds(half, HRPT)],
                    h0_out.at[pl.ds(base + half, HRPT)])
    plsc.subcore_barrier()

    def body(j, carry):
        pltpu.async_copy(y_sh.at[src_v.at[j]], rows_v, sem).wait()
        pltpu.sync_copy(rows_v, acc.at[dst_v.at[j]], add=True)
        return carry

    lax.fori_loop(0, NCH, body, 0)
    plsc.subcore_barrier()
    pltpu.sync_copy(acc.at[pl.ds(base, RPT)],
                    part_out.at[cid, pl.ds(base, RPT)])


GB = 8
BR = NP // GB

_row = pl.BlockSpec((BR, C), lambda i: (i, 0))
_rowx = pl.BlockSpec((BR, FEATS), lambda i: (i, 0))
_smem = pl.BlockSpec(memory_space=pltpu.SMEM)


def _full(shape):
    return pl.BlockSpec(shape, lambda i: tuple(0 for _ in shape))


def _dinv_body(part_ref, dinvb_ref, dinv2b_ref):
    i = pl.program_id(0)
    deg = part_ref[0] + part_ref[1] + 1.0
    dinv = lax.rsqrt(jnp.maximum(deg, 1.0))
    row = i * BR + lax.broadcasted_iota(jnp.int32, (BR, C), 0)
    dinv = dinv * (row < N).astype(jnp.float32)
    dinvb_ref[...] = dinv
    dinv2b_ref[...] = dinv * dinv


_tc_dinv = pl.pallas_call(
    _dinv_body,
    grid=(GB,),
    in_specs=[pl.BlockSpec((NC, BR, C), lambda i: (0, i, 0))],
    out_specs=(_row, _row),
    out_shape=(jax.ShapeDtypeStruct((NP, C), jnp.float32),
               jax.ShapeDtypeStruct((NP, C), jnp.float32)),
)


def _mlp_body(x_ref, w1_ref, b1_ref, w2_ref, b2_ref, dinvb_ref,
              cur_ref, y_ref):
    h1 = lax.dot_general(x_ref[...], w1_ref[...], (((1,), (1,)), ((), ())),
                         preferred_element_type=jnp.float32)
    h1 = jnp.maximum(h1 + b1_ref[...], 0.0)
    h = lax.dot_general(h1, w2_ref[...], (((1,), (1,)), ((), ())),
                        preferred_element_type=jnp.float32)
    h = h + b2_ref[...]
    cur_ref[...] = h
    y_ref[...] = h * dinvb_ref[...]


_tc_mlp = pl.pallas_call(
    _mlp_body,
    grid=(GB,),
    in_specs=[_rowx, _full((HIDDEN, FEATS)), _full((1, HIDDEN)),
              _full((C, HIDDEN)), _full((1, C)), _row],
    out_specs=(_row, _row),
    out_shape=(jax.ShapeDtypeStruct((NP, C), jnp.float32),
               jax.ShapeDtypeStruct((NP, C), jnp.float32)),
)


def _combine_body(part_ref, cur_ref, h0_ref, dinvb_ref, dinv2b_ref, d_ref,
                  ncur_ref, nh0_ref, ny_ref):
    s = part_ref[0] + part_ref[1]
    conv = dinvb_ref[...] * s + dinv2b_ref[...] * cur_ref[...]
    ncur_ref[...] = conv
    nh0_ref[...] = h0_ref[...] + d_ref[0, 0] * conv
    ny_ref[...] = dinvb_ref[...] * conv


_tc_combine = pl.pallas_call(
    _combine_body,
    grid=(GB,),
    in_specs=[pl.BlockSpec((NC, BR, C), lambda i: (0, i, 0)),
              _row, _row, _row, _row, _smem],
    out_specs=(_row, _row, _row),
    out_shape=(jax.ShapeDtypeStruct((NP, C), jnp.float32),
               jax.ShapeDtypeStruct((NP, C), jnp.float32),
               jax.ShapeDtypeStruct((NP, C), jnp.float32)),
)


def _attn_body(x_ref, h0_ref, a1x_ref, ba1_ref, u_ref, v_ref, a2_ref,
               ba2_ref, dinvb_ref, cur_ref, y_ref):
    xa = lax.dot_general(x_ref[...], a1x_ref[...], (((1,), (1,)), ((), ())),
                         preferred_element_type=jnp.float32)
    xa = xa + ba1_ref[...]
    z = h0_ref[...] * (1.0 / DSUM)
    ba2 = ba2_ref[0, 0]
    for c in range(C):
        t = jnp.maximum(xa + z[:, c:c + 1] * u_ref[...] + v_ref[c:c + 1, :],
                        0.0)
        sc = lax.dot_general(t, a2_ref[...], (((1,), (0,)), ((), ())),
                             preferred_element_type=jnp.float32)
        col = sc[:, 0:1] + ba2
        cur_ref[:, c:c + 1] = col
        y_ref[:, c:c + 1] = col * dinvb_ref[:, c:c + 1]


_tc_attn = pl.pallas_call(
    _attn_body,
    grid=(GB,),
    in_specs=[_rowx, _row, _full((HID2P, FEATS)), _full((1, HID2P)),
              _full((1, HID2P)), _full((C, HID2P)), _full((HID2P, 8)),
              _smem, _row],
    out_specs=(_row, _row),
    out_shape=(jax.ShapeDtypeStruct((NP, C), jnp.float32),
               jax.ShapeDtypeStruct((NP, C), jnp.float32)),
)


def _final_body(h0_ref, scl_ref, out_ref):
    out_ref[...] = h0_ref[...] * (scl_ref[0, 0] * (1.0 / DSUM))


_tc_final = pl.pallas_call(
    _final_body,
    grid=(GB,),
    in_specs=[_row, _smem],
    out_specs=_row,
    out_shape=jax.ShapeDtypeStruct((NP, C), jnp.float32),
)


def kernel(x, edges, classes, W1, b1, W2, b2, A1, ba1, A2, ba2):
    f32 = jnp.float32
    x = x.astype(f32)
    src = edges[0].astype(jnp.int32)
    dst = edges[1].astype(jnp.int32)

    pad = EPAD - E
    tail = jnp.full((NW, K, CHUNK), N, jnp.int32)
    src_t = jnp.concatenate([
        jnp.concatenate([src, jnp.full((pad,), N, jnp.int32)]).reshape(
            NW, NCH, CHUNK), tail], axis=1)
    dst_t = jnp.concatenate([
        jnp.concatenate([dst, jnp.full((pad,), N, jnp.int32)]).reshape(
            NW, NCH, CHUNK), tail], axis=1)

    xp = jnp.pad(x, ((0, NP - N), (0, 0)))
    zeros_rpt = jnp.zeros((RPT, C), f32)
    ones_chunk = jnp.ones((CHUNK, C), f32)

    part = _sc_degree(dst_t, ones_chunk, zeros_rpt)
    dinvb, dinv2b = _tc_dinv(part)

    b1r = b1.astype(f32).reshape(1, HIDDEN)
    b2r = b2.astype(f32).reshape(1, C)
    cur, y = _tc_mlp(xp, W1.astype(f32), b1r, W2.astype(f32), b2r, dinvb)
    d_arr = [jnp.full((1, 1), d, f32) for d in DIFF]
    d_vec = [jnp.full((C,), d, f32) for d in DIFF]

    def diffuse(cur, y):
        h0 = cur
        part = _sc_conv(src_t, dst_t, y, zeros_rpt)
        for l in range(DEPTH - 1):
            part, cur, h0 = _sc_round(src_t, dst_t, part, cur, h0, dinvb,
                                      dinv2b, zeros_rpt, d_vec[l])
        cur, h0, y = _tc_combine(part, cur, h0, dinvb, dinv2b,
                                 d_arr[DEPTH - 1])
        return h0

    h0 = diffuse(cur, y)

    A1f = A1.astype(f32)
    hp = HID2P - HID2
    a1x = jnp.pad(A1f[:, 1 + C:], ((0, hp), (0, 0)))
    ba1p = jnp.pad(ba1.astype(f32), (0, hp)).reshape(1, HID2P)
    up = jnp.pad(A1f[:, 0], (0, hp)).reshape(1, HID2P)
    vp = jnp.pad(A1f[:, 1:1 + C].T, ((0, 0), (0, hp)))
    a2p = jnp.pad(A2.astype(f32).T, ((0, hp), (0, 7)))
    ba2r = ba2.astype(f32).reshape(1, 1)

    cur, y = _tc_attn(xp, h0, a1x, ba1p, up, vp, a2p, ba2r, dinvb)
    h0 = diffuse(cur, y)

    scl = (jnp.asarray(classes, f32) / C).reshape(1, 1)
    out = _tc_final(h0, scl)
    return out[:N]


# speedup vs baseline: 2.8047x; 1.0138x over previous
import functools

import jax
import jax.numpy as jnp
from jax import lax
from jax.experimental import pallas as pl
from jax.experimental.pallas import tpu as pltpu
from jax.experimental.pallas import tpu_sc as plsc

N = 10000
E = 320000
FEATS = 128
HIDDEN = 64
C = 16
DEPTH = 10
HID2 = 147
HID2P = 256

NC = 2
NS = 16
NW = NC * NS
CHUNK = 128
K = 4
NCHT = 84
NCH = NCHT - K
EPAD = NW * NCH * CHUNK
NP = 10112
RPT = NP // NS

DIFF = [0.9 ** l for l in range(1, DEPTH + 1)]
DSUM = 1.0 + sum(DIFF)

_MESH = plsc.VectorSubcoreMesh(core_axis_name="c", subcore_axis_name="s")
_SC_PARAMS = pltpu.CompilerParams(use_tc_tiling_on_sc=False)


@functools.partial(
    pl.kernel,
    mesh=_MESH,
    out_type=jax.ShapeDtypeStruct((NC, NP, C), jnp.float32),
    scratch_types=[
        pltpu.VMEM((NCHT, CHUNK), jnp.int32),
        pltpu.VMEM((CHUNK, C), jnp.float32),
        pltpu.VMEM_SHARED((NP, C), jnp.float32),
    ],
    compiler_params=_SC_PARAMS,
)
def _sc_degree(dst_hbm, ones_hbm, zeros_hbm, part_hbm, dst_v, ones_v, acc):
    cid = lax.axis_index("c")
    sid = lax.axis_index("s")
    wid = sid * NC + cid
    pltpu.sync_copy(zeros_hbm, acc.at[pl.ds(sid * RPT, RPT)])
    pltpu.sync_copy(dst_hbm.at[wid], dst_v)
    pltpu.sync_copy(ones_hbm, ones_v)
    plsc.subcore_barrier()

    def body(j, carry):
        pltpu.sync_copy(ones_v, acc.at[dst_v.at[j]], add=True)
        return carry

    lax.fori_loop(0, NCHT, body, 0)
    plsc.subcore_barrier()
    pltpu.sync_copy(acc.at[pl.ds(sid * RPT, RPT)],
                    part_hbm.at[cid, pl.ds(sid * RPT, RPT)])


@functools.partial(
    pl.kernel,
    mesh=_MESH,
    out_type=jax.ShapeDtypeStruct((NC, NP, C), jnp.float32),
    scratch_types=[
        pltpu.VMEM((NCHT, CHUNK), jnp.int32),
        pltpu.VMEM((NCHT, CHUNK), jnp.int32),
        pltpu.VMEM((CHUNK, C), jnp.float32),
        pltpu.VMEM_SHARED((NP, C), jnp.float32),
        pltpu.VMEM_SHARED((NP, C), jnp.float32),
        pltpu.SemaphoreType.DMA,
    ],
    compiler_params=_SC_PARAMS,
)
def _sc_conv(src_hbm, dst_hbm, y_hbm, zeros_hbm, part_hbm,
             src_v, dst_v, rows_v, y_sh, acc, sem):
    cid = lax.axis_index("c")
    sid = lax.axis_index("s")
    wid = sid * NC + cid
    base = sid * RPT
    pltpu.sync_copy(zeros_hbm, acc.at[pl.ds(base, RPT)])
    pltpu.sync_copy(y_hbm.at[pl.ds(base, RPT)], y_sh.at[pl.ds(base, RPT)])
    pltpu.sync_copy(src_hbm.at[wid], src_v)
    pltpu.sync_copy(dst_hbm.at[wid], dst_v)
    plsc.subcore_barrier()

    def body(j, carry):
        pltpu.async_copy(y_sh.at[src_v.at[j]], rows_v, sem).wait()
        pltpu.sync_copy(rows_v, acc.at[dst_v.at[j]], add=True)
        return carry

    lax.fori_loop(0, NCH, body, 0)
    plsc.subcore_barrier()
    pltpu.sync_copy(acc.at[pl.ds(base, RPT)],
                    part_hbm.at[cid, pl.ds(base, RPT)])


HRPT = RPT // 2


@functools.partial(
    pl.kernel,
    mesh=_MESH,
    out_type=(jax.ShapeDtypeStruct((NC, NP, C), jnp.float32),
              jax.ShapeDtypeStruct((NP, C), jnp.float32),
              jax.ShapeDtypeStruct((NP, C), jnp.float32)),
    scratch_types=[
        pltpu.VMEM((NCHT, CHUNK), jnp.int32),
        pltpu.VMEM((NCHT, CHUNK), jnp.int32),
        pltpu.VMEM((CHUNK, C), jnp.float32),
        pltpu.VMEM((RPT, C), jnp.float32),
        pltpu.VMEM((RPT, C), jnp.float32),
        pltpu.VMEM((RPT, C), jnp.float32),
        pltpu.VMEM((RPT, C), jnp.float32),
        pltpu.VMEM((RPT, C), jnp.float32),
        pltpu.VMEM((RPT, C), jnp.float32),
        pltpu.VMEM((RPT, C), jnp.float32),
        pltpu.VMEM((C,), jnp.float32),
        pltpu.VMEM((CHUNK, C), jnp.float32),
        pltpu.VMEM_SHARED((NP, C), jnp.float32),
        pltpu.VMEM_SHARED((NP, C), jnp.float32),
        pltpu.SemaphoreType.DMA,
        pltpu.SemaphoreType.DMA,
    ],
    compiler_params=_SC_PARAMS,
)
def _sc_round(src_hbm, dst_hbm, part_in, cur_in, h0_in, dinvb_hbm,
              dinv2b_hbm, zeros_hbm, dvec_hbm,
              part_out, cur_out, h0_out,
              src_v, dst_v, rows_v, p0_v, p1_v, cur_v, h0_v, db_v, d2_v,
              yb_v, dd_v, rows2_v, y_sh, acc, sem, sem2):
    cid = lax.axis_index("c")
    sid = lax.axis_index("s")
    wid = sid * NC + cid
    base = sid * RPT
    pltpu.sync_copy(zeros_hbm, acc.at[pl.ds(base, RPT)])
    pltpu.sync_copy(src_hbm.at[wid], src_v)
    pltpu.sync_copy(dst_hbm.at[wid], dst_v)
    pltpu.sync_copy(part_in.at[0, pl.ds(base, RPT)], p0_v)
    pltpu.sync_copy(part_in.at[1, pl.ds(base, RPT)], p1_v)
    pltpu.sync_copy(cur_in.at[pl.ds(base, RPT)], cur_v)
    pltpu.sync_copy(h0_in.at[pl.ds(base, RPT)], h0_v)
    pltpu.sync_copy(dinvb_hbm.at[pl.ds(base, RPT)], db_v)
    pltpu.sync_copy(dinv2b_hbm.at[pl.ds(base, RPT)], d2_v)
    pltpu.sync_copy(dvec_hbm, dd_v)
    dd = dd_v[...]

    def rowbody(i, carry):
        conv = db_v[i] * (p0_v[i] + p1_v[i]) + d2_v[i] * cur_v[i]
        cur_v[i] = conv
        h0_v[i] = h0_v[i] + dd * conv
        yb_v[i] = db_v[i] * conv
        return carry

    lax.fori_loop(0, RPT, rowbody, 0, unroll=4)
    pltpu.sync_copy(yb_v, y_sh.at[pl.ds(base, RPT)])
    half = cid * HRPT
    pltpu.sync_copy(cur_v.at[pl.ds(half, HRPT)],
                    cur_out.at[pl.ds(base + half, HRPT)])
    pltpu.sync_copy(h0_v.at[pl.ds(half, HRPT)],
                    h0_out.at[pl.ds(base + half, HRPT)])
    plsc.subcore_barrier()

    def body(t, carry):
        j = 2 * t
        cp0 = pltpu.async_copy(y_sh.at[src_v.at[j]], rows_v, sem)
        cp1 = pltpu.async_copy(y_sh.at[src_v.at[j + 1]], rows2_v, sem2)
        cp0.wait()
        pltpu.sync_copy(rows_v, acc.at[dst_v.at[j]], add=True)
        cp1.wait()
        pltpu.sync_copy(rows2_v, acc.at[dst_v.at[j + 1]], add=True)
        return carry

    lax.fori_loop(0, NCH // 2, body, 0)
    plsc.subcore_barrier()
    pltpu.sync_copy(acc.at[pl.ds(base, RPT)],
                    part_out.at[cid, pl.ds(base, RPT)])


GB = 8
BR = NP // GB

_row = pl.BlockSpec((BR, C), lambda i: (i, 0))
_rowx = pl.BlockSpec((BR, FEATS), lambda i: (i, 0))
_smem = pl.BlockSpec(memory_space=pltpu.SMEM)


def _full(shape):
    return pl.BlockSpec(shape, lambda i: tuple(0 for _ in shape))


def _dinv_body(part_ref, dinvb_ref, dinv2b_ref):
    i = pl.program_id(0)
    deg = part_ref[0] + part_ref[1] + 1.0
    dinv = lax.rsqrt(jnp.maximum(deg, 1.0))
    row = i * BR + lax.broadcasted_iota(jnp.int32, (BR, C), 0)
    dinv = dinv * (row < N).astype(jnp.float32)
    dinvb_ref[...] = dinv
    dinv2b_ref[...] = dinv * dinv


_tc_dinv = pl.pallas_call(
    _dinv_body,
    grid=(GB,),
    in_specs=[pl.BlockSpec((NC, BR, C), lambda i: (0, i, 0))],
    out_specs=(_row, _row),
    out_shape=(jax.ShapeDtypeStruct((NP, C), jnp.float32),
               jax.ShapeDtypeStruct((NP, C), jnp.float32)),
)


def _mlp_body(x_ref, w1_ref, b1_ref, w2_ref, b2_ref, dinvb_ref,
              cur_ref, y_ref):
    h1 = lax.dot_general(x_ref[...], w1_ref[...], (((1,), (1,)), ((), ())),
                         preferred_element_type=jnp.float32)
    h1 = jnp.maximum(h1 + b1_ref[...], 0.0)
    h = lax.dot_general(h1, w2_ref[...], (((1,), (1,)), ((), ())),
                        preferred_element_type=jnp.float32)
    h = h + b2_ref[...]
    cur_ref[...] = h
    y_ref[...] = h * dinvb_ref[...]


_tc_mlp = pl.pallas_call(
    _mlp_body,
    grid=(GB,),
    in_specs=[_rowx, _full((HIDDEN, FEATS)), _full((1, HIDDEN)),
              _full((C, HIDDEN)), _full((1, C)), _row],
    out_specs=(_row, _row),
    out_shape=(jax.ShapeDtypeStruct((NP, C), jnp.float32),
               jax.ShapeDtypeStruct((NP, C), jnp.float32)),
)


def _combine_body(part_ref, cur_ref, h0_ref, dinvb_ref, dinv2b_ref, d_ref,
                  ncur_ref, nh0_ref, ny_ref):
    s = part_ref[0] + part_ref[1]
    conv = dinvb_ref[...] * s + dinv2b_ref[...] * cur_ref[...]
    ncur_ref[...] = conv
    nh0_ref[...] = h0_ref[...] + d_ref[0, 0] * conv
    ny_ref[...] = dinvb_ref[...] * conv


_tc_combine = pl.pallas_call(
    _combine_body,
    grid=(GB,),
    in_specs=[pl.BlockSpec((NC, BR, C), lambda i: (0, i, 0)),
              _row, _row, _row, _row, _smem],
    out_specs=(_row, _row, _row),
    out_shape=(jax.ShapeDtypeStruct((NP, C), jnp.float32),
               jax.ShapeDtypeStruct((NP, C), jnp.float32),
               jax.ShapeDtypeStruct((NP, C), jnp.float32)),
)


def _attn_body(x_ref, h0_ref, a1x_ref, ba1_ref, u_ref, v_ref, a2_ref,
               ba2_ref, dinvb_ref, cur_ref, y_ref):
    xa = lax.dot_general(x_ref[...], a1x_ref[...], (((1,), (1,)), ((), ())),
                         preferred_element_type=jnp.float32)
    xa = xa + ba1_ref[...]
    z = h0_ref[...] * (1.0 / DSUM)
    ba2 = ba2_ref[0, 0]
    for c in range(C):
        t = jnp.maximum(xa + z[:, c:c + 1] * u_ref[...] + v_ref[c:c + 1, :],
                        0.0)
        sc = lax.dot_general(t, a2_ref[...], (((1,), (0,)), ((), ())),
                             preferred_element_type=jnp.float32)
        col = sc[:, 0:1] + ba2
        cur_ref[:, c:c + 1] = col
        y_ref[:, c:c + 1] = col * dinvb_ref[:, c:c + 1]


_tc_attn = pl.pallas_call(
    _attn_body,
    grid=(GB,),
    in_specs=[_rowx, _row, _full((HID2P, FEATS)), _full((1, HID2P)),
              _full((1, HID2P)), _full((C, HID2P)), _full((HID2P, 8)),
              _smem, _row],
    out_specs=(_row, _row),
    out_shape=(jax.ShapeDtypeStruct((NP, C), jnp.float32),
               jax.ShapeDtypeStruct((NP, C), jnp.float32)),
)


def _final_body(h0_ref, scl_ref, out_ref):
    out_ref[...] = h0_ref[...] * (scl_ref[0, 0] * (1.0 / DSUM))


_tc_final = pl.pallas_call(
    _final_body,
    grid=(GB,),
    in_specs=[_row, _smem],
    out_specs=_row,
    out_shape=jax.ShapeDtypeStruct((NP, C), jnp.float32),
)


def kernel(x, edges, classes, W1, b1, W2, b2, A1, ba1, A2, ba2):
    f32 = jnp.float32
    x = x.astype(f32)
    src = edges[0].astype(jnp.int32)
    dst = edges[1].astype(jnp.int32)

    pad = EPAD - E
    tail = jnp.full((NW, K, CHUNK), N, jnp.int32)
    src_t = jnp.concatenate([
        jnp.concatenate([src, jnp.full((pad,), N, jnp.int32)]).reshape(
            NW, NCH, CHUNK), tail], axis=1)
    dst_t = jnp.concatenate([
        jnp.concatenate([dst, jnp.full((pad,), N, jnp.int32)]).reshape(
            NW, NCH, CHUNK), tail], axis=1)

    xp = jnp.pad(x, ((0, NP - N), (0, 0)))
    zeros_rpt = jnp.zeros((RPT, C), f32)
    ones_chunk = jnp.ones((CHUNK, C), f32)

    part = _sc_degree(dst_t, ones_chunk, zeros_rpt)
    dinvb, dinv2b = _tc_dinv(part)

    b1r = b1.astype(f32).reshape(1, HIDDEN)
    b2r = b2.astype(f32).reshape(1, C)
    cur, y = _tc_mlp(xp, W1.astype(f32), b1r, W2.astype(f32), b2r, dinvb)
    d_arr = [jnp.full((1, 1), d, f32) for d in DIFF]
    d_vec = [jnp.full((C,), d, f32) for d in DIFF]

    def diffuse(cur, y):
        h0 = cur
        part = _sc_conv(src_t, dst_t, y, zeros_rpt)
        for l in range(DEPTH - 1):
            part, cur, h0 = _sc_round(src_t, dst_t, part, cur, h0, dinvb,
                                      dinv2b, zeros_rpt, d_vec[l])
        cur, h0, y = _tc_combine(part, cur, h0, dinvb, dinv2b,
                                 d_arr[DEPTH - 1])
        return h0

    h0 = diffuse(cur, y)

    A1f = A1.astype(f32)
    hp = HID2P - HID2
    a1x = jnp.pad(A1f[:, 1 + C:], ((0, hp), (0, 0)))
    ba1p = jnp.pad(ba1.astype(f32), (0, hp)).reshape(1, HID2P)
    up = jnp.pad(A1f[:, 0], (0, hp)).reshape(1, HID2P)
    vp = jnp.pad(A1f[:, 1:1 + C].T, ((0, 0), (0, hp)))
    a2p = jnp.pad(A2.astype(f32).T, ((0, hp), (0, 7)))
    ba2r = ba2.astype(f32).reshape(1, 1)

    cur, y = _tc_attn(xp, h0, a1x, ba1p, up, vp, a2p, ba2r, dinvb)
    h0 = diffuse(cur, y)

    scl = (jnp.asarray(classes, f32) / C).reshape(1, 1)
    out = _tc_final(h0, scl)
    return out[:N]


# speedup vs baseline: 2.8851x; 1.0287x over previous
import functools

import jax
import jax.numpy as jnp
from jax import lax
from jax.experimental import pallas as pl
from jax.experimental.pallas import tpu as pltpu
from jax.experimental.pallas import tpu_sc as plsc

N = 10000
E = 320000
FEATS = 128
HIDDEN = 64
C = 16
DEPTH = 10
HID2 = 147
HID2P = 256

NC = 2
NS = 16
NW = NC * NS
CHUNK = 128
K = 4
NCHT = 84
NCH = NCHT - K
EPAD = NW * NCH * CHUNK
NP = 10112
RPT = NP // NS

DIFF = [0.9 ** l for l in range(1, DEPTH + 1)]
DSUM = 1.0 + sum(DIFF)

_MESH = plsc.VectorSubcoreMesh(core_axis_name="c", subcore_axis_name="s")
_SC_PARAMS = pltpu.CompilerParams(use_tc_tiling_on_sc=False)


@functools.partial(
    pl.kernel,
    mesh=_MESH,
    out_type=jax.ShapeDtypeStruct((NC, NP, C), jnp.float32),
    scratch_types=[
        pltpu.VMEM((NCHT, CHUNK), jnp.int32),
        pltpu.VMEM((CHUNK, C), jnp.float32),
        pltpu.VMEM_SHARED((NP, C), jnp.float32),
    ],
    compiler_params=_SC_PARAMS,
)
def _sc_degree(dst_hbm, ones_hbm, zeros_hbm, part_hbm, dst_v, ones_v, acc):
    cid = lax.axis_index("c")
    sid = lax.axis_index("s")
    wid = sid * NC + cid
    pltpu.sync_copy(zeros_hbm, acc.at[pl.ds(sid * RPT, RPT)])
    pltpu.sync_copy(dst_hbm.at[wid], dst_v)
    pltpu.sync_copy(ones_hbm, ones_v)
    plsc.subcore_barrier()

    def body(j, carry):
        pltpu.sync_copy(ones_v, acc.at[dst_v.at[j]], add=True)
        return carry

    lax.fori_loop(0, NCHT, body, 0)
    plsc.subcore_barrier()
    pltpu.sync_copy(acc.at[pl.ds(sid * RPT, RPT)],
                    part_hbm.at[cid, pl.ds(sid * RPT, RPT)])


@functools.partial(
    pl.kernel,
    mesh=_MESH,
    out_type=jax.ShapeDtypeStruct((NC, NP, C), jnp.float32),
    scratch_types=[
        pltpu.VMEM((NCHT, CHUNK), jnp.int32),
        pltpu.VMEM((NCHT, CHUNK), jnp.int32),
        pltpu.VMEM((CHUNK, C), jnp.float32),
        pltpu.VMEM_SHARED((NP, C), jnp.float32),
        pltpu.VMEM_SHARED((NP, C), jnp.float32),
        pltpu.SemaphoreType.DMA,
    ],
    compiler_params=_SC_PARAMS,
)
def _sc_conv(src_hbm, dst_hbm, y_hbm, zeros_hbm, part_hbm,
             src_v, dst_v, rows_v, y_sh, acc, sem):
    cid = lax.axis_index("c")
    sid = lax.axis_index("s")
    wid = sid * NC + cid
    base = sid * RPT
    pltpu.sync_copy(zeros_hbm, acc.at[pl.ds(base, RPT)])
    pltpu.sync_copy(y_hbm.at[pl.ds(base, RPT)], y_sh.at[pl.ds(base, RPT)])
    pltpu.sync_copy(src_hbm.at[wid], src_v)
    pltpu.sync_copy(dst_hbm.at[wid], dst_v)
    plsc.subcore_barrier()

    def body(j, carry):
        pltpu.async_copy(y_sh.at[src_v.at[j]], rows_v, sem).wait()
        pltpu.sync_copy(rows_v, acc.at[dst_v.at[j]], add=True)
        return carry

    lax.fori_loop(0, NCH, body, 0)
    plsc.subcore_barrier()
    pltpu.sync_copy(acc.at[pl.ds(base, RPT)],
                    part_hbm.at[cid, pl.ds(base, RPT)])


HRPT = RPT // 2


@functools.partial(
    pl.kernel,
    mesh=_MESH,
    out_type=(jax.ShapeDtypeStruct((NC, NP, C), jnp.float32),
              jax.ShapeDtypeStruct((NP, C), jnp.float32),
              jax.ShapeDtypeStruct((NP, C), jnp.float32)),
    scratch_types=[
        pltpu.VMEM((NCHT, CHUNK), jnp.int32),
        pltpu.VMEM((NCHT, CHUNK), jnp.int32),
        pltpu.VMEM((CHUNK, C), jnp.float32),
        pltpu.VMEM((RPT, C), jnp.float32),
        pltpu.VMEM((RPT, C), jnp.float32),
        pltpu.VMEM((RPT, C), jnp.float32),
        pltpu.VMEM((RPT, C), jnp.float32),
        pltpu.VMEM((RPT, C), jnp.float32),
        pltpu.VMEM((RPT, C), jnp.float32),
        pltpu.VMEM((RPT, C), jnp.float32),
        pltpu.VMEM((C,), jnp.float32),
        pltpu.VMEM((CHUNK, C), jnp.float32),
        pltpu.VMEM_SHARED((NP, C), jnp.float32),
        pltpu.VMEM_SHARED((NP, C), jnp.float32),
        pltpu.SemaphoreType.DMA,
        pltpu.SemaphoreType.DMA,
    ],
    compiler_params=_SC_PARAMS,
)
def _sc_round(src_hbm, dst_hbm, part_in, cur_in, h0_in, dinvb_hbm,
              dinv2b_hbm, zeros_hbm, dvec_hbm,
              part_out, cur_out, h0_out,
              src_v, dst_v, rows_v, p0_v, p1_v, cur_v, h0_v, db_v, d2_v,
              yb_v, dd_v, rows2_v, y_sh, acc, sem, sem2):
    cid = lax.axis_index("c")
    sid = lax.axis_index("s")
    wid = sid * NC + cid
    base = sid * RPT
    pltpu.sync_copy(zeros_hbm, acc.at[pl.ds(base, RPT)])
    pltpu.sync_copy(src_hbm.at[wid], src_v)
    pltpu.sync_copy(dst_hbm.at[wid], dst_v)
    pltpu.sync_copy(part_in.at[0, pl.ds(base, RPT)], p0_v)
    pltpu.sync_copy(part_in.at[1, pl.ds(base, RPT)], p1_v)
    pltpu.sync_copy(cur_in.at[pl.ds(base, RPT)], cur_v)
    pltpu.sync_copy(h0_in.at[pl.ds(base, RPT)], h0_v)
    pltpu.sync_copy(dinvb_hbm.at[pl.ds(base, RPT)], db_v)
    pltpu.sync_copy(dinv2b_hbm.at[pl.ds(base, RPT)], d2_v)
    pltpu.sync_copy(dvec_hbm, dd_v)
    dd = dd_v[...]

    def rowbody(i, carry):
        conv = db_v[i] * (p0_v[i] + p1_v[i]) + d2_v[i] * cur_v[i]
        cur_v[i] = conv
        h0_v[i] = h0_v[i] + dd * conv
        yb_v[i] = db_v[i] * conv
        return carry

    lax.fori_loop(0, RPT, rowbody, 0, unroll=4)
    pltpu.sync_copy(yb_v, y_sh.at[pl.ds(base, RPT)])
    half = cid * HRPT
    pltpu.sync_copy(cur_v.at[pl.ds(half, HRPT)],
                    cur_out.at[pl.ds(base + half, HRPT)])
    pltpu.sync_copy(h0_v.at[pl.ds(half, HRPT)],
                    h0_out.at[pl.ds(base + half, HRPT)])
    plsc.subcore_barrier()

    def body(t, carry):
        j = 2 * t
        cp0 = pltpu.async_copy(y_sh.at[src_v.at[j]], rows_v, sem)
        cp1 = pltpu.async_copy(y_sh.at[src_v.at[j + 1]], rows2_v, sem2)
        cp0.wait()
        pltpu.sync_copy(rows_v, acc.at[dst_v.at[j]], add=True)
        cp1.wait()
        pltpu.sync_copy(rows2_v, acc.at[dst_v.at[j + 1]], add=True)
        return carry

    lax.fori_loop(0, NCH // 2, body, 0)
    plsc.subcore_barrier()
    pltpu.sync_copy(acc.at[pl.ds(base, RPT)],
                    part_out.at[cid, pl.ds(base, RPT)])


GB = 8
BR = NP // GB

_row = pl.BlockSpec((BR, C), lambda i: (i, 0))
_rowx = pl.BlockSpec((BR, FEATS), lambda i: (i, 0))
_smem = pl.BlockSpec(memory_space=pltpu.SMEM)


def _full(shape):
    return pl.BlockSpec(shape, lambda i: tuple(0 for _ in shape))


def _mlp_body(x_ref, w1_ref, b1_ref, w2_ref, b2_ref, part_ref,
              cur_ref, y_ref, dinvb_ref, dinv2b_ref):
    i = pl.program_id(0)
    deg = part_ref[0] + part_ref[1] + 1.0
    dinv = lax.rsqrt(jnp.maximum(deg, 1.0))
    row = i * BR + lax.broadcasted_iota(jnp.int32, (BR, C), 0)
    dinv = dinv * (row < N).astype(jnp.float32)
    dinvb_ref[...] = dinv
    dinv2b_ref[...] = dinv * dinv
    h1 = lax.dot_general(x_ref[...], w1_ref[...], (((1,), (1,)), ((), ())),
                         preferred_element_type=jnp.float32)
    h1 = jnp.maximum(h1 + b1_ref[...], 0.0)
    h = lax.dot_general(h1, w2_ref[...], (((1,), (1,)), ((), ())),
                        preferred_element_type=jnp.float32)
    h = h + b2_ref[...]
    cur_ref[...] = h
    y_ref[...] = h * dinv


_tc_mlp = pl.pallas_call(
    _mlp_body,
    grid=(GB,),
    in_specs=[_rowx, _full((HIDDEN, FEATS)), _full((1, HIDDEN)),
              _full((C, HIDDEN)), _full((1, C)),
              pl.BlockSpec((NC, BR, C), lambda i: (0, i, 0))],
    out_specs=(_row, _row, _row, _row),
    out_shape=(jax.ShapeDtypeStruct((NP, C), jnp.float32),
               jax.ShapeDtypeStruct((NP, C), jnp.float32),
               jax.ShapeDtypeStruct((NP, C), jnp.float32),
               jax.ShapeDtypeStruct((NP, C), jnp.float32)),
)


def _attn_body(x_ref, part_ref, c_ref, h0_ref, a1x_ref, ba1_ref, u_ref,
               v_ref, a2_ref, ba2_ref, dinvb_ref, dinv2b_ref,
               cur_ref, y_ref):
    conv = (dinvb_ref[...] * (part_ref[0] + part_ref[1])
            + dinv2b_ref[...] * c_ref[...])
    z = (h0_ref[...] + DIFF[DEPTH - 1] * conv) * (1.0 / DSUM)
    xa = lax.dot_general(x_ref[...], a1x_ref[...], (((1,), (1,)), ((), ())),
                         preferred_element_type=jnp.float32)
    xa = xa + ba1_ref[...]
    ba2 = ba2_ref[0, 0]
    for c in range(C):
        t = jnp.maximum(xa + z[:, c:c + 1] * u_ref[...] + v_ref[c:c + 1, :],
                        0.0)
        sc = lax.dot_general(t, a2_ref[...], (((1,), (0,)), ((), ())),
                             preferred_element_type=jnp.float32)
        col = sc[:, 0:1] + ba2
        cur_ref[:, c:c + 1] = col
        y_ref[:, c:c + 1] = col * dinvb_ref[:, c:c + 1]


_tc_attn = pl.pallas_call(
    _attn_body,
    grid=(GB,),
    in_specs=[_rowx, pl.BlockSpec((NC, BR, C), lambda i: (0, i, 0)),
              _row, _row, _full((HID2P, FEATS)), _full((1, HID2P)),
              _full((1, HID2P)), _full((C, HID2P)), _full((HID2P, 8)),
              _smem, _row, _row],
    out_specs=(_row, _row),
    out_shape=(jax.ShapeDtypeStruct((NP, C), jnp.float32),
               jax.ShapeDtypeStruct((NP, C), jnp.float32)),
)


def _final_body(part_ref, c_ref, h0_ref, dinvb_ref, dinv2b_ref, scl_ref,
                out_ref):
    conv = (dinvb_ref[...] * (part_ref[0] + part_ref[1])
            + dinv2b_ref[...] * c_ref[...])
    h0 = h0_ref[...] + DIFF[DEPTH - 1] * conv
    out_ref[...] = h0 * (scl_ref[0, 0] * (1.0 / DSUM))


_tc_final = pl.pallas_call(
    _final_body,
    grid=(GB,),
    in_specs=[pl.BlockSpec((NC, BR, C), lambda i: (0, i, 0)),
              _row, _row, _row, _row, _smem],
    out_specs=_row,
    out_shape=jax.ShapeDtypeStruct((NP, C), jnp.float32),
)


def kernel(x, edges, classes, W1, b1, W2, b2, A1, ba1, A2, ba2):
    f32 = jnp.float32
    x = x.astype(f32)
    src = edges[0].astype(jnp.int32)
    dst = edges[1].astype(jnp.int32)

    pad = EPAD - E
    tail = jnp.full((NW, K, CHUNK), N, jnp.int32)
    src_t = jnp.concatenate([
        jnp.concatenate([src, jnp.full((pad,), N, jnp.int32)]).reshape(
            NW, NCH, CHUNK), tail], axis=1)
    dst_t = jnp.concatenate([
        jnp.concatenate([dst, jnp.full((pad,), N, jnp.int32)]).reshape(
            NW, NCH, CHUNK), tail], axis=1)

    xp = jnp.pad(x, ((0, NP - N), (0, 0)))
    zeros_rpt = jnp.zeros((RPT, C), f32)
    ones_chunk = jnp.ones((CHUNK, C), f32)

    part_deg = _sc_degree(dst_t, ones_chunk, zeros_rpt)

    b1r = b1.astype(f32).reshape(1, HIDDEN)
    b2r = b2.astype(f32).reshape(1, C)
    cur, y, dinvb, dinv2b = _tc_mlp(xp, W1.astype(f32), b1r,
                                    W2.astype(f32), b2r, part_deg)
    d_vec = [jnp.full((C,), d, f32) for d in DIFF]

    def diffuse(cur, y):
        h0 = cur
        part = _sc_conv(src_t, dst_t, y, zeros_rpt)
        for l in range(DEPTH - 1):
            part, cur, h0 = _sc_round(src_t, dst_t, part, cur, h0, dinvb,
                                      dinv2b, zeros_rpt, d_vec[l])
        return part, cur, h0

    part, cur, h0 = diffuse(cur, y)

    A1f = A1.astype(f32)
    hp = HID2P - HID2
    a1x = jnp.pad(A1f[:, 1 + C:], ((0, hp), (0, 0)))
    ba1p = jnp.pad(ba1.astype(f32), (0, hp)).reshape(1, HID2P)
    up = jnp.pad(A1f[:, 0], (0, hp)).reshape(1, HID2P)
    vp = jnp.pad(A1f[:, 1:1 + C].T, ((0, 0), (0, hp)))
    a2p = jnp.pad(A2.astype(f32).T, ((0, hp), (0, 7)))
    ba2r = ba2.astype(f32).reshape(1, 1)

    cur, y = _tc_attn(xp, part, cur, h0, a1x, ba1p, up, vp, a2p, ba2r,
                      dinvb, dinv2b)
    part, cur, h0 = diffuse(cur, y)

    scl = (jnp.asarray(classes, f32) / C).reshape(1, 1)
    out = _tc_final(part, cur, h0, dinvb, dinv2b, scl)
    return out[:N]


# speedup vs baseline: 3.1197x; 1.0813x over previous
import functools

import jax
import jax.numpy as jnp
from jax import lax
from jax.experimental import pallas as pl
from jax.experimental.pallas import tpu as pltpu
from jax.experimental.pallas import tpu_sc as plsc

N = 10000
E = 320000
FEATS = 128
HIDDEN = 64
C = 16
DEPTH = 10
HID2 = 147
HID2P = 256

NC = 2
NS = 16
NW = NC * NS
CHUNK = 128
K = 4
NCHT = 84
NCH = NCHT - K
EPAD = NW * NCH * CHUNK
NP = 10112
RPT = NP // NS

DIFF = [0.9 ** l for l in range(1, DEPTH + 1)]
DSUM = 1.0 + sum(DIFF)

_MESH = plsc.VectorSubcoreMesh(core_axis_name="c", subcore_axis_name="s")
_SC_PARAMS = pltpu.CompilerParams(use_tc_tiling_on_sc=False)


@functools.partial(
    pl.kernel,
    mesh=_MESH,
    out_type=jax.ShapeDtypeStruct((NC, NP, C), jnp.float32),
    scratch_types=[
        pltpu.VMEM((NCHT, CHUNK), jnp.int32),
        pltpu.VMEM((CHUNK, C), jnp.float32),
        pltpu.VMEM_SHARED((NP, C), jnp.float32),
        pltpu.SemaphoreType.DMA,
        pltpu.SemaphoreType.DMA,
    ],
    compiler_params=_SC_PARAMS,
)
def _sc_degree(dst_hbm, ones_hbm, zeros_hbm, part_hbm, dst_v, ones_v, acc,
               s0, s1):
    cid = lax.axis_index("c")
    sid = lax.axis_index("s")
    wid = sid * NC + cid
    pltpu.sync_copy(zeros_hbm, acc.at[pl.ds(sid * RPT, RPT)])
    pltpu.sync_copy(dst_hbm.at[wid], dst_v)
    pltpu.sync_copy(ones_hbm, ones_v)
    plsc.subcore_barrier()

    def body(t, carry):
        j = 2 * t
        cp0 = pltpu.async_copy(ones_v, acc.at[dst_v.at[j]], s0, add=True)
        cp1 = pltpu.async_copy(ones_v, acc.at[dst_v.at[j + 1]], s1, add=True)
        cp0.wait()
        cp1.wait()
        return carry

    lax.fori_loop(0, NCHT // 2, body, 0)
    plsc.subcore_barrier()
    pltpu.sync_copy(acc.at[pl.ds(sid * RPT, RPT)],
                    part_hbm.at[cid, pl.ds(sid * RPT, RPT)])


@functools.partial(
    pl.kernel,
    mesh=_MESH,
    out_type=jax.ShapeDtypeStruct((NC, NP, C), jnp.float32),
    scratch_types=[
        pltpu.VMEM((NCHT, CHUNK), jnp.int32),
        pltpu.VMEM((NCHT, CHUNK), jnp.int32),
        pltpu.VMEM((CHUNK, C), jnp.float32),
        pltpu.VMEM_SHARED((NP, C), jnp.float32),
        pltpu.VMEM_SHARED((NP, C), jnp.float32),
        pltpu.SemaphoreType.DMA,
    ],
    compiler_params=_SC_PARAMS,
)
def _sc_conv(src_hbm, dst_hbm, y_hbm, zeros_hbm, part_hbm,
             src_v, dst_v, rows_v, y_sh, acc, sem):
    cid = lax.axis_index("c")
    sid = lax.axis_index("s")
    wid = sid * NC + cid
    base = sid * RPT
    pltpu.sync_copy(zeros_hbm, acc.at[pl.ds(base, RPT)])
    pltpu.sync_copy(y_hbm.at[pl.ds(base, RPT)], y_sh.at[pl.ds(base, RPT)])
    pltpu.sync_copy(src_hbm.at[wid], src_v)
    pltpu.sync_copy(dst_hbm.at[wid], dst_v)
    plsc.subcore_barrier()

    def body(j, carry):
        pltpu.async_copy(y_sh.at[src_v.at[j]], rows_v, sem).wait()
        pltpu.sync_copy(rows_v, acc.at[dst_v.at[j]], add=True)
        return carry

    lax.fori_loop(0, NCH, body, 0)
    plsc.subcore_barrier()
    pltpu.sync_copy(acc.at[pl.ds(base, RPT)],
                    part_hbm.at[cid, pl.ds(base, RPT)])


HRPT = RPT // 2


@functools.partial(
    pl.kernel,
    mesh=_MESH,
    out_type=(jax.ShapeDtypeStruct((NC, NP, C), jnp.float32),
              jax.ShapeDtypeStruct((NP, C), jnp.float32),
              jax.ShapeDtypeStruct((NP, C), jnp.float32)),
    scratch_types=[
        pltpu.VMEM((NCHT, CHUNK), jnp.int32),
        pltpu.VMEM((NCHT, CHUNK), jnp.int32),
        pltpu.VMEM((CHUNK, C), jnp.float32),
        pltpu.VMEM((RPT, C), jnp.float32),
        pltpu.VMEM((RPT, C), jnp.float32),
        pltpu.VMEM((RPT, C), jnp.float32),
        pltpu.VMEM((RPT, C), jnp.float32),
        pltpu.VMEM((RPT, C), jnp.float32),
        pltpu.VMEM((RPT, C), jnp.float32),
        pltpu.VMEM((RPT, C), jnp.float32),
        pltpu.VMEM((C,), jnp.float32),
        pltpu.VMEM((CHUNK, C), jnp.float32),
        pltpu.VMEM_SHARED((NP, C), jnp.float32),
        pltpu.VMEM_SHARED((NP, C), jnp.float32),
        pltpu.SemaphoreType.DMA,
        pltpu.SemaphoreType.DMA,
    ],
    compiler_params=_SC_PARAMS,
)
def _sc_round(src_hbm, dst_hbm, part_in, cur_in, h0_in, dinvb_hbm,
              dinv2b_hbm, zeros_hbm, dvec_hbm,
              part_out, cur_out, h0_out,
              src_v, dst_v, rows_v, p0_v, p1_v, cur_v, h0_v, db_v, d2_v,
              yb_v, dd_v, rows2_v, y_sh, acc, sem, sem2):
    cid = lax.axis_index("c")
    sid = lax.axis_index("s")
    wid = sid * NC + cid
    base = sid * RPT
    cps = [
        pltpu.async_copy(zeros_hbm, acc.at[pl.ds(base, RPT)], sem),
        pltpu.async_copy(src_hbm.at[wid], src_v, sem),
        pltpu.async_copy(dst_hbm.at[wid], dst_v, sem),
        pltpu.async_copy(part_in.at[0, pl.ds(base, RPT)], p0_v, sem),
        pltpu.async_copy(part_in.at[1, pl.ds(base, RPT)], p1_v, sem),
        pltpu.async_copy(cur_in.at[pl.ds(base, RPT)], cur_v, sem),
        pltpu.async_copy(h0_in.at[pl.ds(base, RPT)], h0_v, sem),
        pltpu.async_copy(dinvb_hbm.at[pl.ds(base, RPT)], db_v, sem),
        pltpu.async_copy(dinv2b_hbm.at[pl.ds(base, RPT)], d2_v, sem),
        pltpu.async_copy(dvec_hbm, dd_v, sem),
    ]
    for cp in cps:
        cp.wait()
    dd = dd_v[...]

    def rowbody(i, carry):
        conv = db_v[i] * (p0_v[i] + p1_v[i]) + d2_v[i] * cur_v[i]
        cur_v[i] = conv
        h0_v[i] = h0_v[i] + dd * conv
        yb_v[i] = db_v[i] * conv
        return carry

    lax.fori_loop(0, RPT, rowbody, 0, unroll=4)
    pltpu.sync_copy(yb_v, y_sh.at[pl.ds(base, RPT)])
    half = cid * HRPT
    pltpu.sync_copy(cur_v.at[pl.ds(half, HRPT)],
                    cur_out.at[pl.ds(base + half, HRPT)])
    pltpu.sync_copy(h0_v.at[pl.ds(half, HRPT)],
                    h0_out.at[pl.ds(base + half, HRPT)])
    plsc.subcore_barrier()

    def body(t, carry):
        j = 2 * t
        cp0 = pltpu.async_copy(y_sh.at[src_v.at[j]], rows_v, sem)
        cp1 = pltpu.async_copy(y_sh.at[src_v.at[j + 1]], rows2_v, sem2)
        cp0.wait()
        pltpu.sync_copy(rows_v, acc.at[dst_v.at[j]], add=True)
        cp1.wait()
        pltpu.sync_copy(rows2_v, acc.at[dst_v.at[j + 1]], add=True)
        return carry

    lax.fori_loop(0, NCH // 2, body, 0)
    plsc.subcore_barrier()
    pltpu.sync_copy(acc.at[pl.ds(base, RPT)],
                    part_out.at[cid, pl.ds(base, RPT)])


GB = 8
BR = NP // GB

_row = pl.BlockSpec((BR, C), lambda i: (i, 0))
_rowx = pl.BlockSpec((BR, FEATS), lambda i: (i, 0))
_smem = pl.BlockSpec(memory_space=pltpu.SMEM)


def _full(shape):
    return pl.BlockSpec(shape, lambda i: tuple(0 for _ in shape))


def _mlp_body(x_ref, w1_ref, b1_ref, w2_ref, b2_ref, part_ref,
              cur_ref, y_ref, dinvb_ref, dinv2b_ref):
    i = pl.program_id(0)
    deg = part_ref[0] + part_ref[1] + 1.0
    dinv = lax.rsqrt(jnp.maximum(deg, 1.0))
    row = i * BR + lax.broadcasted_iota(jnp.int32, (BR, C), 0)
    dinv = dinv * (row < N).astype(jnp.float32)
    dinvb_ref[...] = dinv
    dinv2b_ref[...] = dinv * dinv
    h1 = lax.dot_general(x_ref[...], w1_ref[...], (((1,), (1,)), ((), ())),
                         preferred_element_type=jnp.float32)
    h1 = jnp.maximum(h1 + b1_ref[...], 0.0)
    h = lax.dot_general(h1, w2_ref[...], (((1,), (1,)), ((), ())),
                        preferred_element_type=jnp.float32)
    h = h + b2_ref[...]
    cur_ref[...] = h
    y_ref[...] = h * dinv


_tc_mlp = pl.pallas_call(
    _mlp_body,
    grid=(GB,),
    in_specs=[_rowx, _full((HIDDEN, FEATS)), _full((1, HIDDEN)),
              _full((C, HIDDEN)), _full((1, C)),
              pl.BlockSpec((NC, BR, C), lambda i: (0, i, 0))],
    out_specs=(_row, _row, _row, _row),
    out_shape=(jax.ShapeDtypeStruct((NP, C), jnp.float32),
               jax.ShapeDtypeStruct((NP, C), jnp.float32),
               jax.ShapeDtypeStruct((NP, C), jnp.float32),
               jax.ShapeDtypeStruct((NP, C), jnp.float32)),
)


def _attn_body(x_ref, part_ref, c_ref, h0_ref, a1x_ref, ba1_ref, u_ref,
               v_ref, a2_ref, ba2_ref, dinvb_ref, dinv2b_ref,
               cur_ref, y_ref):
    conv = (dinvb_ref[...] * (part_ref[0] + part_ref[1])
            + dinv2b_ref[...] * c_ref[...])
    z = (h0_ref[...] + DIFF[DEPTH - 1] * conv) * (1.0 / DSUM)
    xa = lax.dot_general(x_ref[...], a1x_ref[...], (((1,), (1,)), ((), ())),
                         preferred_element_type=jnp.float32)
    xa = xa + ba1_ref[...]
    ba2 = ba2_ref[0, 0]
    for c in range(C):
        t = jnp.maximum(xa + z[:, c:c + 1] * u_ref[...] + v_ref[c:c + 1, :],
                        0.0)
        sc = lax.dot_general(t, a2_ref[...], (((1,), (0,)), ((), ())),
                             preferred_element_type=jnp.float32)
        col = sc[:, 0:1] + ba2
        cur_ref[:, c:c + 1] = col
        y_ref[:, c:c + 1] = col * dinvb_ref[:, c:c + 1]


_tc_attn = pl.pallas_call(
    _attn_body,
    grid=(GB,),
    in_specs=[_rowx, pl.BlockSpec((NC, BR, C), lambda i: (0, i, 0)),
              _row, _row, _full((HID2P, FEATS)), _full((1, HID2P)),
              _full((1, HID2P)), _full((C, HID2P)), _full((HID2P, 8)),
              _smem, _row, _row],
    out_specs=(_row, _row),
    out_shape=(jax.ShapeDtypeStruct((NP, C), jnp.float32),
               jax.ShapeDtypeStruct((NP, C), jnp.float32)),
)


def _final_body(part_ref, c_ref, h0_ref, dinvb_ref, dinv2b_ref, scl_ref,
                out_ref):
    conv = (dinvb_ref[...] * (part_ref[0] + part_ref[1])
            + dinv2b_ref[...] * c_ref[...])
    h0 = h0_ref[...] + DIFF[DEPTH - 1] * conv
    out_ref[...] = h0 * (scl_ref[0, 0] * (1.0 / DSUM))


_tc_final = pl.pallas_call(
    _final_body,
    grid=(GB,),
    in_specs=[pl.BlockSpec((NC, BR, C), lambda i: (0, i, 0)),
              _row, _row, _row, _row, _smem],
    out_specs=_row,
    out_shape=jax.ShapeDtypeStruct((NP, C), jnp.float32),
)


def kernel(x, edges, classes, W1, b1, W2, b2, A1, ba1, A2, ba2):
    f32 = jnp.float32
    x = x.astype(f32)
    src = edges[0].astype(jnp.int32)
    dst = edges[1].astype(jnp.int32)

    pad = EPAD - E
    tail = jnp.full((NW, K, CHUNK), N, jnp.int32)
    src_t = jnp.concatenate([
        jnp.concatenate([src, jnp.full((pad,), N, jnp.int32)]).reshape(
            NW, NCH, CHUNK), tail], axis=1)
    dst_t = jnp.concatenate([
        jnp.concatenate([dst, jnp.full((pad,), N, jnp.int32)]).reshape(
            NW, NCH, CHUNK), tail], axis=1)

    xp = jnp.pad(x, ((0, NP - N), (0, 0)))
    zeros_rpt = jnp.zeros((RPT, C), f32)
    ones_chunk = jnp.ones((CHUNK, C), f32)

    part_deg = _sc_degree(dst_t, ones_chunk, zeros_rpt)

    b1r = b1.astype(f32).reshape(1, HIDDEN)
    b2r = b2.astype(f32).reshape(1, C)
    cur, y, dinvb, dinv2b = _tc_mlp(xp, W1.astype(f32), b1r,
                                    W2.astype(f32), b2r, part_deg)
    d_vec = [jnp.full((C,), d, f32) for d in DIFF]

    def diffuse(cur, y):
        h0 = cur
        part = _sc_conv(src_t, dst_t, y, zeros_rpt)
        for l in range(DEPTH - 1):
            part, cur, h0 = _sc_round(src_t, dst_t, part, cur, h0, dinvb,
                                      dinv2b, zeros_rpt, d_vec[l])
        return part, cur, h0

    part, cur, h0 = diffuse(cur, y)

    A1f = A1.astype(f32)
    hp = HID2P - HID2
    a1x = jnp.pad(A1f[:, 1 + C:], ((0, hp), (0, 0)))
    ba1p = jnp.pad(ba1.astype(f32), (0, hp)).reshape(1, HID2P)
    up = jnp.pad(A1f[:, 0], (0, hp)).reshape(1, HID2P)
    vp = jnp.pad(A1f[:, 1:1 + C].T, ((0, 0), (0, hp)))
    a2p = jnp.pad(A2.astype(f32).T, ((0, hp), (0, 7)))
    ba2r = ba2.astype(f32).reshape(1, 1)

    cur, y = _tc_attn(xp, part, cur, h0, a1x, ba1p, up, vp, a2p, ba2r,
                      dinvb, dinv2b)
    part, cur, h0 = diffuse(cur, y)

    scl = (jnp.asarray(classes, f32) / C).reshape(1, 1)
    out = _tc_final(part, cur, h0, dinvb, dinv2b, scl)
    return out[:N]


# speedup vs baseline: 3.3159x; 1.0629x over previous
import functools

import jax
import jax.numpy as jnp
from jax import lax
from jax.experimental import pallas as pl
from jax.experimental.pallas import tpu as pltpu
from jax.experimental.pallas import tpu_sc as plsc

N = 10000
E = 320000
FEATS = 128
HIDDEN = 64
C = 16
DEPTH = 10
HID2 = 147
HID2P = 256

NC = 2
NS = 16
NW = NC * NS
CHUNK = 128
K = 4
NCHT = 84
NCH = NCHT - K
EPAD = NW * NCH * CHUNK
NP = 10112
RPT = NP // NS

DIFF = [0.9 ** l for l in range(1, DEPTH + 1)]
DSUM = 1.0 + sum(DIFF)

_MESH = plsc.VectorSubcoreMesh(core_axis_name="c", subcore_axis_name="s")
_SC_PARAMS = pltpu.CompilerParams(use_tc_tiling_on_sc=False)


@functools.partial(
    pl.kernel,
    mesh=_MESH,
    out_type=jax.ShapeDtypeStruct((NC, NP, C), jnp.float32),
    scratch_types=[
        pltpu.VMEM((NCHT, CHUNK), jnp.int32),
        pltpu.VMEM((CHUNK, C), jnp.float32),
        pltpu.VMEM_SHARED((NP, C), jnp.float32),
        pltpu.SemaphoreType.DMA,
        pltpu.SemaphoreType.DMA,
    ],
    compiler_params=_SC_PARAMS,
)
def _sc_degree(dst_hbm, ones_hbm, zeros_hbm, part_hbm, dst_v, ones_v, acc,
               s0, s1):
    cid = lax.axis_index("c")
    sid = lax.axis_index("s")
    wid = sid * NC + cid
    pltpu.sync_copy(zeros_hbm, acc.at[pl.ds(sid * RPT, RPT)])
    pltpu.sync_copy(dst_hbm.at[wid], dst_v)
    pltpu.sync_copy(ones_hbm, ones_v)
    plsc.subcore_barrier()

    def body(t, carry):
        j = 2 * t
        cp0 = pltpu.async_copy(ones_v, acc.at[dst_v.at[j]], s0, add=True)
        cp1 = pltpu.async_copy(ones_v, acc.at[dst_v.at[j + 1]], s1, add=True)
        cp0.wait()
        cp1.wait()
        return carry

    lax.fori_loop(0, NCHT // 2, body, 0)
    plsc.subcore_barrier()
    pltpu.sync_copy(acc.at[pl.ds(sid * RPT, RPT)],
                    part_hbm.at[cid, pl.ds(sid * RPT, RPT)])


@functools.partial(
    pl.kernel,
    mesh=_MESH,
    out_type=jax.ShapeDtypeStruct((NC, NP, C), jnp.float32),
    scratch_types=[
        pltpu.VMEM((NCHT, CHUNK), jnp.int32),
        pltpu.VMEM((NCHT, CHUNK), jnp.int32),
        pltpu.VMEM((CHUNK, C), jnp.float32),
        pltpu.VMEM_SHARED((NP, C), jnp.float32),
        pltpu.VMEM_SHARED((NP, C), jnp.float32),
        pltpu.SemaphoreType.DMA,
    ],
    compiler_params=_SC_PARAMS,
)
def _sc_conv(src_hbm, dst_hbm, y_hbm, zeros_hbm, part_hbm,
             src_v, dst_v, rows_v, y_sh, acc, sem):
    cid = lax.axis_index("c")
    sid = lax.axis_index("s")
    wid = sid * NC + cid
    base = sid * RPT
    cps = [
        pltpu.async_copy(zeros_hbm, acc.at[pl.ds(base, RPT)], sem),
        pltpu.async_copy(y_hbm.at[pl.ds(base, RPT)],
                         y_sh.at[pl.ds(base, RPT)], sem),
        pltpu.async_copy(src_hbm.at[wid], src_v, sem),
        pltpu.async_copy(dst_hbm.at[wid], dst_v, sem),
    ]
    for cp in cps:
        cp.wait()
    plsc.subcore_barrier()

    def body(j, carry):
        pltpu.async_copy(y_sh.at[src_v.at[j]], rows_v, sem).wait()
        pltpu.sync_copy(rows_v, acc.at[dst_v.at[j]], add=True)
        return carry

    lax.fori_loop(0, NCH, body, 0)
    plsc.subcore_barrier()
    pltpu.sync_copy(acc.at[pl.ds(base, RPT)],
                    part_hbm.at[cid, pl.ds(base, RPT)])


HRPT = RPT // 2


@functools.partial(
    pl.kernel,
    mesh=_MESH,
    out_type=(jax.ShapeDtypeStruct((NC, NP, C), jnp.float32),
              jax.ShapeDtypeStruct((NP, C), jnp.float32),
              jax.ShapeDtypeStruct((NP, C), jnp.float32)),
    scratch_types=[
        pltpu.VMEM((NCHT, CHUNK), jnp.int32),
        pltpu.VMEM((NCHT, CHUNK), jnp.int32),
        pltpu.VMEM((CHUNK, C), jnp.float32),
        pltpu.VMEM((RPT, C), jnp.float32),
        pltpu.VMEM((RPT, C), jnp.float32),
        pltpu.VMEM((RPT, C), jnp.float32),
        pltpu.VMEM((RPT, C), jnp.float32),
        pltpu.VMEM((RPT, C), jnp.float32),
        pltpu.VMEM((RPT, C), jnp.float32),
        pltpu.VMEM((RPT, C), jnp.float32),
        pltpu.VMEM((C,), jnp.float32),
        pltpu.VMEM((CHUNK, C), jnp.float32),
        pltpu.VMEM((CHUNK, C), jnp.float32),
        pltpu.VMEM((CHUNK, C), jnp.float32),
        pltpu.VMEM_SHARED((NP, C), jnp.float32),
        pltpu.VMEM_SHARED((NP, C), jnp.float32),
        pltpu.SemaphoreType.DMA,
        pltpu.SemaphoreType.DMA,
        pltpu.SemaphoreType.DMA,
        pltpu.SemaphoreType.DMA,
        pltpu.SemaphoreType.DMA,
        pltpu.SemaphoreType.DMA,
        pltpu.SemaphoreType.DMA,
        pltpu.SemaphoreType.DMA,
    ],
    compiler_params=_SC_PARAMS,
)
def _sc_round(src_hbm, dst_hbm, part_in, cur_in, h0_in, dinvb_hbm,
              dinv2b_hbm, zeros_hbm, dvec_hbm,
              part_out, cur_out, h0_out,
              src_v, dst_v, rows_v, p0_v, p1_v, cur_v, h0_v, db_v, d2_v,
              yb_v, dd_v, rows2_v, rows3_v, rows4_v, y_sh, acc,
              sem, sem2, sg3, sg4, ss1, ss2, ss3, ss4):
    cid = lax.axis_index("c")
    sid = lax.axis_index("s")
    wid = sid * NC + cid
    base = sid * RPT
    cps = [
        pltpu.async_copy(zeros_hbm, acc.at[pl.ds(base, RPT)], sem),
        pltpu.async_copy(src_hbm.at[wid], src_v, sem),
        pltpu.async_copy(dst_hbm.at[wid], dst_v, sem),
        pltpu.async_copy(part_in.at[0, pl.ds(base, RPT)], p0_v, sem),
        pltpu.async_copy(part_in.at[1, pl.ds(base, RPT)], p1_v, sem),
        pltpu.async_copy(cur_in.at[pl.ds(base, RPT)], cur_v, sem),
        pltpu.async_copy(h0_in.at[pl.ds(base, RPT)], h0_v, sem),
        pltpu.async_copy(dinvb_hbm.at[pl.ds(base, RPT)], db_v, sem),
        pltpu.async_copy(dinv2b_hbm.at[pl.ds(base, RPT)], d2_v, sem),
        pltpu.async_copy(dvec_hbm, dd_v, sem),
    ]
    for cp in cps:
        cp.wait()
    dd = dd_v[...]

    def rowbody(i, carry):
        conv = db_v[i] * (p0_v[i] + p1_v[i]) + d2_v[i] * cur_v[i]
        cur_v[i] = conv
        h0_v[i] = h0_v[i] + dd * conv
        yb_v[i] = db_v[i] * conv
        return carry

    lax.fori_loop(0, RPT, rowbody, 0, unroll=4)
    pltpu.sync_copy(yb_v, y_sh.at[pl.ds(base, RPT)])
    half = cid * HRPT
    pltpu.sync_copy(cur_v.at[pl.ds(half, HRPT)],
                    cur_out.at[pl.ds(base + half, HRPT)])
    pltpu.sync_copy(h0_v.at[pl.ds(half, HRPT)],
                    h0_out.at[pl.ds(base + half, HRPT)])
    plsc.subcore_barrier()

    def body(t, carry):
        j = 4 * t
        bufs = (rows_v, rows2_v, rows3_v, rows4_v)
        gsems = (sem, sem2, sg3, sg4)
        ssems = (ss1, ss2, ss3, ss4)
        gcps = [pltpu.async_copy(y_sh.at[src_v.at[j + b]], bufs[b], gsems[b])
                for b in range(4)]
        scps = []
        for b in range(4):
            gcps[b].wait()
            scps.append(pltpu.async_copy(bufs[b], acc.at[dst_v.at[j + b]],
                                         ssems[b], add=True))
        for cp in scps:
            cp.wait()
        return carry

    lax.fori_loop(0, NCH // 4, body, 0)
    plsc.subcore_barrier()
    pltpu.sync_copy(acc.at[pl.ds(base, RPT)],
                    part_out.at[cid, pl.ds(base, RPT)])


GB = 8
BR = NP // GB

_row = pl.BlockSpec((BR, C), lambda i: (i, 0))
_rowx = pl.BlockSpec((BR, FEATS), lambda i: (i, 0))
_smem = pl.BlockSpec(memory_space=pltpu.SMEM)


def _full(shape):
    return pl.BlockSpec(shape, lambda i: tuple(0 for _ in shape))


def _mlp_body(x_ref, w1_ref, b1_ref, w2_ref, b2_ref, part_ref,
              cur_ref, y_ref, dinvb_ref, dinv2b_ref):
    i = pl.program_id(0)
    deg = part_ref[0] + part_ref[1] + 1.0
    dinv = lax.rsqrt(jnp.maximum(deg, 1.0))
    row = i * BR + lax.broadcasted_iota(jnp.int32, (BR, C), 0)
    dinv = dinv * (row < N).astype(jnp.float32)
    dinvb_ref[...] = dinv
    dinv2b_ref[...] = dinv * dinv
    h1 = lax.dot_general(x_ref[...], w1_ref[...], (((1,), (1,)), ((), ())),
                         preferred_element_type=jnp.float32)
    h1 = jnp.maximum(h1 + b1_ref[...], 0.0)
    h = lax.dot_general(h1, w2_ref[...], (((1,), (1,)), ((), ())),
                        preferred_element_type=jnp.float32)
    h = h + b2_ref[...]
    cur_ref[...] = h
    y_ref[...] = h * dinv


_tc_mlp = pl.pallas_call(
    _mlp_body,
    grid=(GB,),
    in_specs=[_rowx, _full((HIDDEN, FEATS)), _full((1, HIDDEN)),
              _full((C, HIDDEN)), _full((1, C)),
              pl.BlockSpec((NC, BR, C), lambda i: (0, i, 0))],
    out_specs=(_row, _row, _row, _row),
    out_shape=(jax.ShapeDtypeStruct((NP, C), jnp.float32),
               jax.ShapeDtypeStruct((NP, C), jnp.float32),
               jax.ShapeDtypeStruct((NP, C), jnp.float32),
               jax.ShapeDtypeStruct((NP, C), jnp.float32)),
)


def _attn_body(x_ref, part_ref, c_ref, h0_ref, a1x_ref, ba1_ref, u_ref,
               v_ref, a2_ref, ba2_ref, dinvb_ref, dinv2b_ref,
               cur_ref, y_ref):
    conv = (dinvb_ref[...] * (part_ref[0] + part_ref[1])
            + dinv2b_ref[...] * c_ref[...])
    z = (h0_ref[...] + DIFF[DEPTH - 1] * conv) * (1.0 / DSUM)
    xa = lax.dot_general(x_ref[...], a1x_ref[...], (((1,), (1,)), ((), ())),
                         preferred_element_type=jnp.float32)
    xa = xa + ba1_ref[...]
    ba2 = ba2_ref[0, 0]
    for c in range(C):
        t = jnp.maximum(xa + z[:, c:c + 1] * u_ref[...] + v_ref[c:c + 1, :],
                        0.0)
        sc = lax.dot_general(t, a2_ref[...], (((1,), (0,)), ((), ())),
                             preferred_element_type=jnp.float32)
        col = sc[:, 0:1] + ba2
        cur_ref[:, c:c + 1] = col
        y_ref[:, c:c + 1] = col * dinvb_ref[:, c:c + 1]


_tc_attn = pl.pallas_call(
    _attn_body,
    grid=(GB,),
    in_specs=[_rowx, pl.BlockSpec((NC, BR, C), lambda i: (0, i, 0)),
              _row, _row, _full((HID2P, FEATS)), _full((1, HID2P)),
              _full((1, HID2P)), _full((C, HID2P)), _full((HID2P, 8)),
              _smem, _row, _row],
    out_specs=(_row, _row),
    out_shape=(jax.ShapeDtypeStruct((NP, C), jnp.float32),
               jax.ShapeDtypeStruct((NP, C), jnp.float32)),
)


def _final_body(part_ref, c_ref, h0_ref, dinvb_ref, dinv2b_ref, scl_ref,
                out_ref):
    conv = (dinvb_ref[...] * (part_ref[0] + part_ref[1])
            + dinv2b_ref[...] * c_ref[...])
    h0 = h0_ref[...] + DIFF[DEPTH - 1] * conv
    out_ref[...] = h0 * (scl_ref[0, 0] * (1.0 / DSUM))


_tc_final = pl.pallas_call(
    _final_body,
    grid=(GB,),
    in_specs=[pl.BlockSpec((NC, BR, C), lambda i: (0, i, 0)),
              _row, _row, _row, _row, _smem],
    out_specs=_row,
    out_shape=jax.ShapeDtypeStruct((NP, C), jnp.float32),
)


def kernel(x, edges, classes, W1, b1, W2, b2, A1, ba1, A2, ba2):
    f32 = jnp.float32
    x = x.astype(f32)
    src = edges[0].astype(jnp.int32)
    dst = edges[1].astype(jnp.int32)

    pad = EPAD - E
    tail = jnp.full((NW, K, CHUNK), N, jnp.int32)
    src_t = jnp.concatenate([
        jnp.concatenate([src, jnp.full((pad,), N, jnp.int32)]).reshape(
            NW, NCH, CHUNK), tail], axis=1)
    dst_t = jnp.concatenate([
        jnp.concatenate([dst, jnp.full((pad,), N, jnp.int32)]).reshape(
            NW, NCH, CHUNK), tail], axis=1)

    xp = jnp.pad(x, ((0, NP - N), (0, 0)))
    zeros_rpt = jnp.zeros((RPT, C), f32)
    ones_chunk = jnp.ones((CHUNK, C), f32)

    part_deg = _sc_degree(dst_t, ones_chunk, zeros_rpt)

    b1r = b1.astype(f32).reshape(1, HIDDEN)
    b2r = b2.astype(f32).reshape(1, C)
    cur, y, dinvb, dinv2b = _tc_mlp(xp, W1.astype(f32), b1r,
                                    W2.astype(f32), b2r, part_deg)
    d_vec = [jnp.full((C,), d, f32) for d in DIFF]

    def diffuse(cur, y):
        h0 = cur
        part = _sc_conv(src_t, dst_t, y, zeros_rpt)
        for l in range(DEPTH - 1):
            part, cur, h0 = _sc_round(src_t, dst_t, part, cur, h0, dinvb,
                                      dinv2b, zeros_rpt, d_vec[l])
        return part, cur, h0

    part, cur, h0 = diffuse(cur, y)

    A1f = A1.astype(f32)
    hp = HID2P - HID2
    a1x = jnp.pad(A1f[:, 1 + C:], ((0, hp), (0, 0)))
    ba1p = jnp.pad(ba1.astype(f32), (0, hp)).reshape(1, HID2P)
    up = jnp.pad(A1f[:, 0], (0, hp)).reshape(1, HID2P)
    vp = jnp.pad(A1f[:, 1:1 + C].T, ((0, 0), (0, hp)))
    a2p = jnp.pad(A2.astype(f32).T, ((0, hp), (0, 7)))
    ba2r = ba2.astype(f32).reshape(1, 1)

    cur, y = _tc_attn(xp, part, cur, h0, a1x, ba1p, up, vp, a2p, ba2r,
                      dinvb, dinv2b)
    part, cur, h0 = diffuse(cur, y)

    scl = (jnp.asarray(classes, f32) / C).reshape(1, 1)
    out = _tc_final(part, cur, h0, dinvb, dinv2b, scl)
    return out[:N]
